# Initial kernel scaffold; baseline (speedup 1.0000x reference)
import jax
import jax.numpy as jnp
from jax.experimental import pallas as pl


def kernel(x, edges_l6, ev6, edges_l5, ev5, pool_idx, unpool_idx, params):
    raise NotImplementedError("write your pallas kernel here")


# speedup vs baseline: 1.2593x; 1.2593x over previous
import functools

import jax
import jax.numpy as jnp
from jax import lax
from jax.experimental import pallas as pl
from jax.experimental.pallas import tpu as pltpu
from jax.experimental.pallas import tpu_sc as plsc

N6, N5 = 40962, 10242
E6, E5 = 245760, 61440
NW = 32
N6P, NPT6, NPA6 = 41472, 1296, 648
N5P, NPT5, NPA5 = 10752, 336, 168
E6P, E5P = E6 + 128, E5 + 128
BE = 96

f32 = jnp.float32
i32 = jnp.int32


def _sc_mesh():
    return plsc.VectorSubcoreMesh(core_axis_name="c", subcore_axis_name="s",
                                  num_cores=2, num_subcores=16)


@functools.cache
def _make_agg(Np, npt, npa, cout, Ep, be):
    kc = 3 * cout
    npass = npt // npa

    @functools.partial(
        pl.kernel, mesh=_sc_mesh(),
        out_type=jax.ShapeDtypeStruct((Np, cout), f32),
        compiler_params=pltpu.CompilerParams(use_tc_tiling_on_sc=False),
        scratch_types=[
            pltpu.VMEM((be,), i32),
            pltpu.VMEM((be,), i32),
            pltpu.VMEM((3 * be,), f32),
            pltpu.VMEM((be, kc), f32),
            pltpu.VMEM((npa, cout), f32),
            pltpu.VMEM((16,), i32),
            pltpu.SemaphoreType.DMA,
        ],
    )
    def agg(xw, g3, srcs, dsts, rowptr, out,
            idx_v, dst_v, g_v, rows_v, acc_v, rp_v, sem):
        wid = lax.axis_index("s") * 2 + lax.axis_index("c")
        n0 = wid * npt
        iota16 = lax.iota(i32, 16)

        def one_pass(p, _):
            n0p = n0 + p * npa
            pltpu.sync_copy(rowptr.at[pl.ds(n0p, 16)], rp_v)
            e0 = rp_v[pl.ds(0, 16)][0]
            pltpu.sync_copy(rowptr.at[pl.ds(n0p + npa, 16)], rp_v)
            e1 = rp_v[pl.ds(0, 16)][0]

            def zrow(n, _):
                for ccx in range(cout // 16):
                    acc_v[n, pl.ds(ccx * 16, 16)] = jnp.zeros((16,), f32)
                return 0
            lax.fori_loop(0, npa, zrow, 0)

            eb0 = (e0 // 8) * 8
            nb = (e1 - eb0 + be - 1) // be

            def batch(bi, _):
                eb = eb0 + bi * be
                pltpu.sync_copy(srcs.at[pl.ds(eb, be)], idx_v)
                pltpu.sync_copy(dsts.at[pl.ds(eb, be)], dst_v)
                for kk in range(3):
                    pltpu.sync_copy(g3.at[pl.ds(kk * Ep + eb, be)],
                                    g_v.at[pl.ds(kk * be, be)])
                pltpu.async_copy(xw.at[idx_v], rows_v, sem).wait()

                def sub(sb, _):
                    jg = eb + sb * 16 + iota16
                    m = jnp.where((jg >= e0) & (jg < e1),
                                  jnp.float32(1.0), jnp.float32(0.0))
                    dv = jnp.clip(dst_v[pl.ds(sb * 16, 16)] - n0p, 0, npa - 1)
                    g0 = g_v[pl.ds(sb * 16, 16)] * m
                    g1 = g_v[pl.ds(be + sb * 16, 16)] * m
                    g2 = g_v[pl.ds(2 * be + sb * 16, 16)] * m
                    for j2 in range(16):
                        j = sb * 16 + j2
                        ld = dv[j2]
                        a = g0[j2]
                        b = g1[j2]
                        cg = g2[j2]
                        for ccx in range(cout // 16):
                            o = ccx * 16
                            v = (rows_v[j, pl.ds(o, 16)] * a
                                 + rows_v[j, pl.ds(cout + o, 16)] * b
                                 + rows_v[j, pl.ds(2 * cout + o, 16)] * cg)
                            acc_v[ld, pl.ds(o, 16)] = acc_v[ld, pl.ds(o, 16)] + v
                    return 0
                lax.fori_loop(0, be // 16, sub, 0)
                return 0
            lax.fori_loop(0, nb, batch, 0)
            pltpu.sync_copy(acc_v, out.at[pl.ds(n0p, npa)])
            return 0
        lax.fori_loop(0, npass, one_pass, 0)

    return agg


@functools.cache
def _make_pool():
    npt, C = NPT5, 64
    nb_nodes = 8

    @functools.partial(
        pl.kernel, mesh=_sc_mesh(),
        out_type=jax.ShapeDtypeStruct((N5P, C), f32),
        compiler_params=pltpu.CompilerParams(use_tc_tiling_on_sc=False),
        scratch_types=[
            pltpu.VMEM((7 * nb_nodes,), i32),
            pltpu.VMEM((7 * nb_nodes, C), f32),
            pltpu.VMEM((npt, C), f32),
            pltpu.SemaphoreType.DMA,
        ],
    )
    def pool(skip, pidx, out, idx_v, rows_v, out_v, sem):
        wid = lax.axis_index("s") * 2 + lax.axis_index("c")
        n0 = wid * npt

        def batch(bi, _):
            pltpu.sync_copy(pidx.at[pl.ds((n0 + bi * nb_nodes) * 7, 7 * nb_nodes)],
                            idx_v)
            pltpu.async_copy(skip.at[idx_v], rows_v, sem).wait()
            for j in range(nb_nodes):
                for cc in range(C // 16):
                    o = cc * 16
                    v = rows_v[7 * j, pl.ds(o, 16)]
                    for t in range(1, 7):
                        v = jnp.maximum(v, rows_v[7 * j + t, pl.ds(o, 16)])
                    out_v[bi * nb_nodes + j, pl.ds(o, 16)] = v
            return 0
        lax.fori_loop(0, npt // nb_nodes, batch, 0)
        pltpu.sync_copy(out_v, out.at[pl.ds(n0, npt)])

    return pool


@functools.cache
def _make_unpool():
    npt, C = NPT6, 128
    nb_rows = 48

    @functools.partial(
        pl.kernel, mesh=_sc_mesh(),
        out_type=jax.ShapeDtypeStruct((N6P, C), f32),
        compiler_params=pltpu.CompilerParams(use_tc_tiling_on_sc=False),
        scratch_types=[
            pltpu.VMEM((2 * nb_rows,), i32),
            pltpu.VMEM((2 * nb_rows, C), f32),
            pltpu.VMEM((nb_rows, C), f32),
            pltpu.SemaphoreType.DMA,
        ],
    )
    def unpool(h4, uf, out, idx_v, rows_v, o_v, sem):
        wid = lax.axis_index("s") * 2 + lax.axis_index("c")
        n0 = wid * npt

        def batch(bi, _):
            base = n0 + bi * nb_rows
            pltpu.sync_copy(uf.at[pl.ds(base * 2, 2 * nb_rows)], idx_v)
            pltpu.async_copy(h4.at[idx_v], rows_v, sem).wait()

            def row(j, _):
                for cc in range(C // 16):
                    o = cc * 16
                    o_v[j, pl.ds(o, 16)] = (rows_v[2 * j, pl.ds(o, 16)]
                                            + rows_v[2 * j + 1, pl.ds(o, 16)]) * 0.5
                return 0
            lax.fori_loop(0, nb_rows, row, 0)
            pltpu.sync_copy(o_v, out.at[pl.ds(base, nb_rows)])
            return 0
        lax.fori_loop(0, npt // nb_rows, batch, 0)

    return unpool


@functools.cache
def _make_gk(Ep, nconv, blk=2048):
    grid = (pl.cdiv(Ep, blk),)

    def body(par_ref, ev_ref, *out_refs):
        e0 = ev_ref[0:1, :]
        e1 = ev_ref[1:2, :]
        for ic in range(nconv):
            for k in range(3):
                a0 = par_ref[ic * 3 + k, 0]
                a1 = par_ref[ic * 3 + k, 1]
                d0 = e0 - par_ref[ic * 3 + k, 2]
                d1 = e1 - par_ref[ic * 3 + k, 3]
                out_refs[ic][k:k + 1, :] = jnp.exp(a0 * d0 * d0 + a1 * d1 * d1)

    return pl.pallas_call(
        body,
        grid=grid,
        in_specs=[
            pl.BlockSpec(memory_space=pltpu.SMEM),
            pl.BlockSpec((2, blk), lambda i: (0, i)),
        ],
        out_specs=[pl.BlockSpec((3, blk), lambda i: (0, i))] * nconv,
        out_shape=[jax.ShapeDtypeStruct((3, Ep), f32)] * nconv,
    )


@functools.cache
def _make_mm(Np, cin, kc, cout, blk=256):
    def body(x_ref, w_ref, xw_ref, r_ref):
        y = jnp.dot(x_ref[...], w_ref[...], preferred_element_type=f32)
        xw_ref[...] = y[:, :kc]
        r_ref[...] = y[:, kc:]

    return pl.pallas_call(
        body,
        grid=(Np // blk,),
        in_specs=[pl.BlockSpec((blk, cin), lambda i: (i, 0)),
                  pl.BlockSpec((cin, kc + cout), lambda i: (0, 0))],
        out_specs=[pl.BlockSpec((blk, kc), lambda i: (i, 0)),
                   pl.BlockSpec((blk, cout), lambda i: (i, 0))],
        out_shape=[jax.ShapeDtypeStruct((Np, kc), f32),
                   jax.ShapeDtypeStruct((Np, cout), f32)],
    )


@functools.cache
def _make_mmfin(Np, cp, kc, cout, blk=256):
    def body(acc_ref, cnt_ref, r_ref, b_ref, w_ref, xw_ref, r2_ref):
        inv = 1.0 / jnp.maximum(cnt_ref[...], 1.0)
        h = jnp.maximum(acc_ref[...] * inv + r_ref[...] + b_ref[...], 0.0)
        y = jnp.dot(h, w_ref[...], preferred_element_type=f32)
        xw_ref[...] = y[:, :kc]
        r2_ref[...] = y[:, kc:]

    return pl.pallas_call(
        body,
        grid=(Np // blk,),
        in_specs=[pl.BlockSpec((blk, cp), lambda i: (i, 0)),
                  pl.BlockSpec((blk, 1), lambda i: (i, 0)),
                  pl.BlockSpec((blk, cp), lambda i: (i, 0)),
                  pl.BlockSpec((1, cp), lambda i: (0, 0)),
                  pl.BlockSpec((cp, kc + cout), lambda i: (0, 0))],
        out_specs=[pl.BlockSpec((blk, kc), lambda i: (i, 0)),
                   pl.BlockSpec((blk, cout), lambda i: (i, 0))],
        out_shape=[jax.ShapeDtypeStruct((Np, kc), f32),
                   jax.ShapeDtypeStruct((Np, cout), f32)],
    )


@functools.cache
def _make_fin(Np, cp, blk=256):
    def body(acc_ref, cnt_ref, r_ref, b_ref, h_ref):
        inv = 1.0 / jnp.maximum(cnt_ref[...], 1.0)
        h_ref[...] = jnp.maximum(acc_ref[...] * inv + r_ref[...] + b_ref[...], 0.0)

    return pl.pallas_call(
        body,
        grid=(Np // blk,),
        in_specs=[pl.BlockSpec((blk, cp), lambda i: (i, 0)),
                  pl.BlockSpec((blk, 1), lambda i: (i, 0)),
                  pl.BlockSpec((blk, cp), lambda i: (i, 0)),
                  pl.BlockSpec((1, cp), lambda i: (0, 0))],
        out_specs=pl.BlockSpec((blk, cp), lambda i: (i, 0)),
        out_shape=jax.ShapeDtypeStruct((Np, cp), f32),
    )


@functools.cache
def _make_mm2(Np, c1, c2, kc, cout, blk=256):
    def body(a_ref, b_ref, wa_ref, wb_ref, xw_ref, r_ref):
        y = (jnp.dot(a_ref[...], wa_ref[...], preferred_element_type=f32)
             + jnp.dot(b_ref[...], wb_ref[...], preferred_element_type=f32))
        xw_ref[...] = y[:, :kc]
        r_ref[...] = y[:, kc:]

    return pl.pallas_call(
        body,
        grid=(Np // blk,),
        in_specs=[pl.BlockSpec((blk, c1), lambda i: (i, 0)),
                  pl.BlockSpec((blk, c2), lambda i: (i, 0)),
                  pl.BlockSpec((c1, kc + cout), lambda i: (0, 0)),
                  pl.BlockSpec((c2, kc + cout), lambda i: (0, 0))],
        out_specs=[pl.BlockSpec((blk, kc), lambda i: (i, 0)),
                   pl.BlockSpec((blk, cout), lambda i: (i, 0))],
        out_shape=[jax.ShapeDtypeStruct((Np, kc), f32),
                   jax.ShapeDtypeStruct((Np, cout), f32)],
    )


@functools.cache
def _make_final(Np, cp=64, blk=256):
    def body(acc_ref, cnt_ref, r_ref, b_ref, fcw_ref, fcb_ref, o_ref):
        inv = 1.0 / jnp.maximum(cnt_ref[...], 1.0)
        h = jnp.maximum(acc_ref[...] * inv + r_ref[...] + b_ref[...], 0.0)
        lg = jnp.dot(h, fcw_ref[...], preferred_element_type=f32) + fcb_ref[...]
        mx = jnp.max(lg, axis=1, keepdims=True)
        lse = mx + jnp.log(jnp.sum(jnp.exp(lg - mx), axis=1, keepdims=True))
        o_ref[...] = lg - lse

    return pl.pallas_call(
        body,
        grid=(Np // blk,),
        in_specs=[pl.BlockSpec((blk, cp), lambda i: (i, 0)),
                  pl.BlockSpec((blk, 1), lambda i: (i, 0)),
                  pl.BlockSpec((blk, cp), lambda i: (i, 0)),
                  pl.BlockSpec((1, cp), lambda i: (0, 0)),
                  pl.BlockSpec((cp, 2), lambda i: (0, 0)),
                  pl.BlockSpec((1, 2), lambda i: (0, 0))],
        out_specs=pl.BlockSpec((blk, 2), lambda i: (i, 0)),
        out_shape=jax.ShapeDtypeStruct((Np, 2), f32),
    )


def _prep_level(edges, ev, N, Np, Ep):
    src = edges[0].astype(i32)
    dst = edges[1].astype(i32)
    E = src.shape[0]
    perm = jnp.argsort(dst)
    dsts = dst[perm]
    srcs = src[perm]
    evs = ev[perm]
    rowptr = jnp.searchsorted(dsts, jnp.arange(Np + 32, dtype=i32),
                              side="left").astype(i32)
    cnt = (rowptr[1:Np + 1] - rowptr[:Np]).astype(f32).reshape(Np, 1)
    srcs_p = jnp.zeros((Ep,), i32).at[:E].set(srcs)
    dsts_p = jnp.full((Ep,), N - 1, i32).at[:E].set(dsts)
    evT = jnp.zeros((2, Ep), f32).at[:, :E].set(evs.T)
    return srcs_p, dsts_p, evT, rowptr, cnt


def _gpar(ps):
    rows = []
    for p in ps:
        a = -0.5 / (p["sigma"] ** 2 + 1e-8)
        rows.append(jnp.concatenate([a, p["mu"]], axis=1))
    return jnp.concatenate(rows, axis=0)


def _wcat(p):
    return jnp.concatenate([p["W"], p["root"]], axis=1)


def kernel(x, edges_l6, ev6, edges_l5, ev5, pool_idx, unpool_idx, params):
    P = params
    src6, dst6, evT6, rp6, cnt6 = _prep_level(edges_l6, ev6, N6, N6P, E6P)
    src5, dst5, evT5, rp5, cnt5 = _prep_level(edges_l5, ev5, N5, N5P, E5P)

    g6 = _make_gk(E6P, 4)(_gpar([P["c1"], P["c2"], P["c5"], P["c6"]]), evT6)
    g5 = _make_gk(E5P, 2)(_gpar([P["c3"], P["c4"]]), evT5)
    g1, g2, gc5, gc6 = [g.reshape(-1) for g in g6]
    g3, g4 = [g.reshape(-1) for g in g5]

    agg6 = _make_agg(N6P, NPT6, NPA6, 64, E6P, BE)
    agg5 = _make_agg(N5P, NPT5, NPA5, 128, E5P, BE)

    xp = jnp.zeros((N6P, x.shape[1]), f32).at[:N6].set(x)
    b = {k: P[k]["b"].reshape(1, -1) for k in ("c1", "c2", "c3", "c4", "c5", "c6")}

    xw1, r1 = _make_mm(N6P, 32, 192, 64)(xp, _wcat(P["c1"]))
    acc1 = agg6(xw1, g1, src6, dst6, rp6)
    xw2, r2 = _make_mmfin(N6P, 64, 192, 64)(acc1, cnt6, r1, b["c1"], _wcat(P["c2"]))
    acc2 = agg6(xw2, g2, src6, dst6, rp6)
    skip = _make_fin(N6P, 64)(acc2, cnt6, r2, b["c2"])

    pidx = jnp.zeros((N5P * 7,), i32).at[:N5 * 7].set(
        pool_idx.astype(i32).reshape(-1))
    hp = _make_pool()(skip, pidx)

    xw3, r3 = _make_mm(N5P, 64, 384, 128)(hp, _wcat(P["c3"]))
    acc3 = agg5(xw3, g3, src5, dst5, rp5)
    xw4, r4 = _make_mmfin(N5P, 128, 384, 128)(acc3, cnt5, r3, b["c3"], _wcat(P["c4"]))
    acc4 = agg5(xw4, g4, src5, dst5, rp5)
    h4 = _make_fin(N5P, 128)(acc4, cnt5, r4, b["c4"])

    ar5 = jnp.arange(N5, dtype=i32)
    u0 = jnp.concatenate([ar5, unpool_idx[:, 0].astype(i32),
                          jnp.zeros((N6P - N6,), i32)])
    u1 = jnp.concatenate([ar5, unpool_idx[:, 1].astype(i32),
                          jnp.zeros((N6P - N6,), i32)])
    uf = jnp.stack([u0, u1], axis=1).reshape(-1)
    up = _make_unpool()(h4, uf)

    p5 = P["c5"]
    wa = jnp.concatenate([p5["W"][:128], p5["root"][:128]], axis=1)
    wb = jnp.concatenate([p5["W"][128:], p5["root"][128:]], axis=1)
    xw5, r5 = _make_mm2(N6P, 128, 64, 192, 64)(up, skip, wa, wb)
    acc5 = agg6(xw5, gc5, src6, dst6, rp6)
    xw6, r6 = _make_mmfin(N6P, 64, 192, 64)(acc5, cnt6, r5, b["c5"], _wcat(P["c6"]))
    acc6 = agg6(xw6, gc6, src6, dst6, rp6)

    out = _make_final(N6P)(acc6, cnt6, r6, b["c6"], P["fc_w"],
                           P["fc_b"].reshape(1, 2))
    return out[:N6]


# speedup vs baseline: 1.4651x; 1.1634x over previous
import functools

import jax
import jax.numpy as jnp
from jax import lax
from jax.experimental import pallas as pl
from jax.experimental.pallas import tpu as pltpu
from jax.experimental.pallas import tpu_sc as plsc

N6, N5 = 40962, 10242
E6, E5 = 245760, 61440
NW = 32
N6P, NPT6, NPA6 = 41472, 1296, 432
N5P, NPT5, NPA5 = 10752, 336, 112
E6P, E5P = E6 + 128, E5 + 128
BE6, BE5 = 64, 48

f32 = jnp.float32
i32 = jnp.int32


def _sc_mesh():
    return plsc.VectorSubcoreMesh(core_axis_name="c", subcore_axis_name="s",
                                  num_cores=2, num_subcores=16)


@functools.cache
def _make_agg(Np, npt, npa, cout, Ep, be):
    kc = 3 * cout
    npass = npt // npa

    @functools.partial(
        pl.kernel, mesh=_sc_mesh(),
        out_type=jax.ShapeDtypeStruct((Np, cout), f32),
        compiler_params=pltpu.CompilerParams(use_tc_tiling_on_sc=False),
        scratch_types=[
            pltpu.VMEM((be,), i32), pltpu.VMEM((be,), i32),
            pltpu.VMEM((be,), i32), pltpu.VMEM((be,), i32),
            pltpu.VMEM((3 * be,), f32), pltpu.VMEM((3 * be,), f32),
            pltpu.VMEM((be, kc), f32), pltpu.VMEM((be, kc), f32),
            pltpu.VMEM((npa, cout), f32),
            pltpu.VMEM((16,), i32),
            pltpu.SemaphoreType.DMA, pltpu.SemaphoreType.DMA,
            pltpu.SemaphoreType.DMA, pltpu.SemaphoreType.DMA,
            pltpu.SemaphoreType.DMA, pltpu.SemaphoreType.DMA,
        ],
    )
    def agg(xw, g3, srcs, dsts, rowptr, out,
            ix0, ix1, dv0, dv1, gv0, gv1, rw0, rw1, acc_v, rp_v,
            si0, si1, sm0, sm1, sg0, sg1):
        idxb, dstb, gvb, rwb = [ix0, ix1], [dv0, dv1], [gv0, gv1], [rw0, rw1]
        sib, smb, sgb = [si0, si1], [sm0, sm1], [sg0, sg1]
        wid = lax.axis_index("s") * 2 + lax.axis_index("c")
        n0 = wid * npt
        iota16 = lax.iota(i32, 16)

        def one_pass(p, _):
            n0p = n0 + p * npa
            pltpu.sync_copy(rowptr.at[pl.ds(n0p, 16)], rp_v)
            e0 = rp_v[pl.ds(0, 16)][0]
            pltpu.sync_copy(rowptr.at[pl.ds(n0p + npa, 16)], rp_v)
            e1 = rp_v[pl.ds(0, 16)][0]

            def zrow(n, _):
                for ccx in range(cout // 16):
                    acc_v[n, pl.ds(ccx * 16, 16)] = jnp.zeros((16,), f32)
                return 0
            lax.fori_loop(0, npa, zrow, 0)

            eb0 = (e0 // 8) * 8
            nb = (e1 - eb0 + be - 1) // be

            def issue_idx(bi, par):
                pltpu.async_copy(srcs.at[pl.ds(eb0 + bi * be, be)],
                                 idxb[par], sib[par])

            def issue_meta(bi, par):
                eb = eb0 + bi * be
                pltpu.async_copy(dsts.at[pl.ds(eb, be)], dstb[par], smb[par])
                for kk in range(3):
                    pltpu.async_copy(g3.at[pl.ds(kk * Ep + eb, be)],
                                     gvb[par].at[pl.ds(kk * be, be)], smb[par])

            def compute(bi, par):
                ebm = eb0 + bi * be

                def sub(sb, _):
                    base = sb * 16
                    jg = ebm + base + iota16
                    m = jnp.where((jg >= e0) & (jg < e1),
                                  jnp.float32(1.0), jnp.float32(0.0))
                    dv = jnp.clip(dstb[par][pl.ds(base, 16)] - n0p, 0, npa - 1)
                    g0 = gvb[par][pl.ds(base, 16)] * m
                    g1 = gvb[par][pl.ds(be + base, 16)] * m
                    g2 = gvb[par][pl.ds(2 * be + base, 16)] * m
                    rows_v = rwb[par]
                    for j2 in range(16):
                        j = base + j2
                        ld = dv[j2]
                        a = g0[j2]
                        b = g1[j2]
                        cg = g2[j2]
                        for ccx in range(cout // 16):
                            o = ccx * 16
                            v = (rows_v[j, pl.ds(o, 16)] * a
                                 + rows_v[j, pl.ds(cout + o, 16)] * b
                                 + rows_v[j, pl.ds(2 * cout + o, 16)] * cg)
                            acc_v[ld, pl.ds(o, 16)] = acc_v[ld, pl.ds(o, 16)] + v
                    return 0
                lax.fori_loop(0, be // 16, sub, 0)

            @pl.when(nb > 0)
            def _():
                issue_idx(0, 0)
                issue_meta(0, 0)

            def pair(i, _):
                for par in range(2):
                    b = i * 2 + par
                    opar = 1 - par

                    @pl.when(b <= nb)
                    def _():
                        @pl.when(b >= 1)
                        def _():
                            pltpu.make_async_copy(
                                xw.at[pl.ds(0, be)], rwb[opar], sgb[opar]).wait()

                        @pl.when(b < nb)
                        def _():
                            pltpu.make_async_copy(
                                srcs.at[pl.ds(0, be)], idxb[par], sib[par]).wait()
                            pltpu.make_async_copy(
                                dsts.at[pl.ds(0, be)], dstb[par], smb[par]).wait()
                            for kk in range(3):
                                pltpu.make_async_copy(
                                    g3.at[pl.ds(0, be)],
                                    gvb[par].at[pl.ds(kk * be, be)],
                                    smb[par]).wait()
                            pltpu.async_copy(xw.at[idxb[par]], rwb[par], sgb[par])

                        @pl.when(b + 1 < nb)
                        def _():
                            issue_idx(b + 1, opar)

                        @pl.when(b >= 1)
                        def _():
                            compute(b - 1, opar)

                        @pl.when(b + 1 < nb)
                        def _():
                            issue_meta(b + 1, opar)
                return 0
            lax.fori_loop(0, (nb + 2) // 2, pair, 0)
            pltpu.sync_copy(acc_v, out.at[pl.ds(n0p, npa)])
            return 0
        lax.fori_loop(0, npass, one_pass, 0)

    return agg


@functools.cache
def _make_pool():
    npt, C = NPT5, 64
    nb_nodes = 8

    @functools.partial(
        pl.kernel, mesh=_sc_mesh(),
        out_type=jax.ShapeDtypeStruct((N5P, C), f32),
        compiler_params=pltpu.CompilerParams(use_tc_tiling_on_sc=False),
        scratch_types=[
            pltpu.VMEM((7 * nb_nodes,), i32),
            pltpu.VMEM((7 * nb_nodes, C), f32),
            pltpu.VMEM((npt, C), f32),
            pltpu.SemaphoreType.DMA,
        ],
    )
    def pool(skip, pidx, out, idx_v, rows_v, out_v, sem):
        wid = lax.axis_index("s") * 2 + lax.axis_index("c")
        n0 = wid * npt

        def batch(bi, _):
            pltpu.sync_copy(pidx.at[pl.ds((n0 + bi * nb_nodes) * 7, 7 * nb_nodes)],
                            idx_v)
            pltpu.async_copy(skip.at[idx_v], rows_v, sem).wait()
            for j in range(nb_nodes):
                for cc in range(C // 16):
                    o = cc * 16
                    v = rows_v[7 * j, pl.ds(o, 16)]
                    for t in range(1, 7):
                        v = jnp.maximum(v, rows_v[7 * j + t, pl.ds(o, 16)])
                    out_v[bi * nb_nodes + j, pl.ds(o, 16)] = v
            return 0
        lax.fori_loop(0, npt // nb_nodes, batch, 0)
        pltpu.sync_copy(out_v, out.at[pl.ds(n0, npt)])

    return pool


@functools.cache
def _make_unpool():
    npt, C = NPT6, 128
    nb_rows = 48

    @functools.partial(
        pl.kernel, mesh=_sc_mesh(),
        out_type=jax.ShapeDtypeStruct((N6P, C), f32),
        compiler_params=pltpu.CompilerParams(use_tc_tiling_on_sc=False),
        scratch_types=[
            pltpu.VMEM((2 * nb_rows,), i32),
            pltpu.VMEM((2 * nb_rows, C), f32),
            pltpu.VMEM((nb_rows, C), f32),
            pltpu.SemaphoreType.DMA,
        ],
    )
    def unpool(h4, uf, out, idx_v, rows_v, o_v, sem):
        wid = lax.axis_index("s") * 2 + lax.axis_index("c")
        n0 = wid * npt

        def batch(bi, _):
            base = n0 + bi * nb_rows
            pltpu.sync_copy(uf.at[pl.ds(base * 2, 2 * nb_rows)], idx_v)
            pltpu.async_copy(h4.at[idx_v], rows_v, sem).wait()

            def row(j, _):
                for cc in range(C // 16):
                    o = cc * 16
                    o_v[j, pl.ds(o, 16)] = (rows_v[2 * j, pl.ds(o, 16)]
                                            + rows_v[2 * j + 1, pl.ds(o, 16)]) * 0.5
                return 0
            lax.fori_loop(0, nb_rows, row, 0)
            pltpu.sync_copy(o_v, out.at[pl.ds(base, nb_rows)])
            return 0
        lax.fori_loop(0, npt // nb_rows, batch, 0)

    return unpool


@functools.cache
def _make_gk(Ep, nconv, blk=2048):
    grid = (pl.cdiv(Ep, blk),)

    def body(par_ref, ev_ref, *out_refs):
        e0 = ev_ref[0:1, :]
        e1 = ev_ref[1:2, :]
        for ic in range(nconv):
            for k in range(3):
                a0 = par_ref[ic * 3 + k, 0]
                a1 = par_ref[ic * 3 + k, 1]
                d0 = e0 - par_ref[ic * 3 + k, 2]
                d1 = e1 - par_ref[ic * 3 + k, 3]
                out_refs[ic][k:k + 1, :] = jnp.exp(a0 * d0 * d0 + a1 * d1 * d1)

    return pl.pallas_call(
        body,
        grid=grid,
        in_specs=[
            pl.BlockSpec(memory_space=pltpu.SMEM),
            pl.BlockSpec((2, blk), lambda i: (0, i)),
        ],
        out_specs=[pl.BlockSpec((3, blk), lambda i: (0, i))] * nconv,
        out_shape=[jax.ShapeDtypeStruct((3, Ep), f32)] * nconv,
    )


@functools.cache
def _make_mm(Np, cin, kc, cout, blk=256):
    def body(x_ref, w_ref, xw_ref, r_ref):
        y = jnp.dot(x_ref[...], w_ref[...], preferred_element_type=f32)
        xw_ref[...] = y[:, :kc]
        r_ref[...] = y[:, kc:]

    return pl.pallas_call(
        body,
        grid=(Np // blk,),
        in_specs=[pl.BlockSpec((blk, cin), lambda i: (i, 0)),
                  pl.BlockSpec((cin, kc + cout), lambda i: (0, 0))],
        out_specs=[pl.BlockSpec((blk, kc), lambda i: (i, 0)),
                   pl.BlockSpec((blk, cout), lambda i: (i, 0))],
        out_shape=[jax.ShapeDtypeStruct((Np, kc), f32),
                   jax.ShapeDtypeStruct((Np, cout), f32)],
    )


@functools.cache
def _make_mmfin(Np, cp, kc, cout, blk=256):
    def body(acc_ref, cnt_ref, r_ref, b_ref, w_ref, xw_ref, r2_ref):
        inv = 1.0 / jnp.maximum(cnt_ref[...], 1.0)
        h = jnp.maximum(acc_ref[...] * inv + r_ref[...] + b_ref[...], 0.0)
        y = jnp.dot(h, w_ref[...], preferred_element_type=f32)
        xw_ref[...] = y[:, :kc]
        r2_ref[...] = y[:, kc:]

    return pl.pallas_call(
        body,
        grid=(Np // blk,),
        in_specs=[pl.BlockSpec((blk, cp), lambda i: (i, 0)),
                  pl.BlockSpec((blk, 1), lambda i: (i, 0)),
                  pl.BlockSpec((blk, cp), lambda i: (i, 0)),
                  pl.BlockSpec((1, cp), lambda i: (0, 0)),
                  pl.BlockSpec((cp, kc + cout), lambda i: (0, 0))],
        out_specs=[pl.BlockSpec((blk, kc), lambda i: (i, 0)),
                   pl.BlockSpec((blk, cout), lambda i: (i, 0))],
        out_shape=[jax.ShapeDtypeStruct((Np, kc), f32),
                   jax.ShapeDtypeStruct((Np, cout), f32)],
    )


@functools.cache
def _make_fin(Np, cp, blk=256):
    def body(acc_ref, cnt_ref, r_ref, b_ref, h_ref):
        inv = 1.0 / jnp.maximum(cnt_ref[...], 1.0)
        h_ref[...] = jnp.maximum(acc_ref[...] * inv + r_ref[...] + b_ref[...], 0.0)

    return pl.pallas_call(
        body,
        grid=(Np // blk,),
        in_specs=[pl.BlockSpec((blk, cp), lambda i: (i, 0)),
                  pl.BlockSpec((blk, 1), lambda i: (i, 0)),
                  pl.BlockSpec((blk, cp), lambda i: (i, 0)),
                  pl.BlockSpec((1, cp), lambda i: (0, 0))],
        out_specs=pl.BlockSpec((blk, cp), lambda i: (i, 0)),
        out_shape=jax.ShapeDtypeStruct((Np, cp), f32),
    )


@functools.cache
def _make_mm2(Np, c1, c2, kc, cout, blk=256):
    def body(a_ref, b_ref, wa_ref, wb_ref, xw_ref, r_ref):
        y = (jnp.dot(a_ref[...], wa_ref[...], preferred_element_type=f32)
             + jnp.dot(b_ref[...], wb_ref[...], preferred_element_type=f32))
        xw_ref[...] = y[:, :kc]
        r_ref[...] = y[:, kc:]

    return pl.pallas_call(
        body,
        grid=(Np // blk,),
        in_specs=[pl.BlockSpec((blk, c1), lambda i: (i, 0)),
                  pl.BlockSpec((blk, c2), lambda i: (i, 0)),
                  pl.BlockSpec((c1, kc + cout), lambda i: (0, 0)),
                  pl.BlockSpec((c2, kc + cout), lambda i: (0, 0))],
        out_specs=[pl.BlockSpec((blk, kc), lambda i: (i, 0)),
                   pl.BlockSpec((blk, cout), lambda i: (i, 0))],
        out_shape=[jax.ShapeDtypeStruct((Np, kc), f32),
                   jax.ShapeDtypeStruct((Np, cout), f32)],
    )


@functools.cache
def _make_final(Np, cp=64, blk=256):
    def body(acc_ref, cnt_ref, r_ref, b_ref, fcw_ref, fcb_ref, o_ref):
        inv = 1.0 / jnp.maximum(cnt_ref[...], 1.0)
        h = jnp.maximum(acc_ref[...] * inv + r_ref[...] + b_ref[...], 0.0)
        lg = jnp.dot(h, fcw_ref[...], preferred_element_type=f32) + fcb_ref[...]
        mx = jnp.max(lg, axis=1, keepdims=True)
        lse = mx + jnp.log(jnp.sum(jnp.exp(lg - mx), axis=1, keepdims=True))
        o_ref[...] = lg - lse

    return pl.pallas_call(
        body,
        grid=(Np // blk,),
        in_specs=[pl.BlockSpec((blk, cp), lambda i: (i, 0)),
                  pl.BlockSpec((blk, 1), lambda i: (i, 0)),
                  pl.BlockSpec((blk, cp), lambda i: (i, 0)),
                  pl.BlockSpec((1, cp), lambda i: (0, 0)),
                  pl.BlockSpec((cp, 2), lambda i: (0, 0)),
                  pl.BlockSpec((1, 2), lambda i: (0, 0))],
        out_specs=pl.BlockSpec((blk, 2), lambda i: (i, 0)),
        out_shape=jax.ShapeDtypeStruct((Np, 2), f32),
    )


def _prep_level(edges, ev, N, Np, Ep):
    src = edges[0].astype(i32)
    dst = edges[1].astype(i32)
    E = src.shape[0]
    perm = jnp.argsort(dst)
    dsts = dst[perm]
    srcs = src[perm]
    evs = ev[perm]
    rowptr = jnp.searchsorted(dsts, jnp.arange(Np + 32, dtype=i32),
                              side="left").astype(i32)
    cnt = (rowptr[1:Np + 1] - rowptr[:Np]).astype(f32).reshape(Np, 1)
    srcs_p = jnp.zeros((Ep,), i32).at[:E].set(srcs)
    dsts_p = jnp.full((Ep,), N - 1, i32).at[:E].set(dsts)
    evT = jnp.zeros((2, Ep), f32).at[:, :E].set(evs.T)
    return srcs_p, dsts_p, evT, rowptr, cnt


def _gpar(ps):
    rows = []
    for p in ps:
        a = -0.5 / (p["sigma"] ** 2 + 1e-8)
        rows.append(jnp.concatenate([a, p["mu"]], axis=1))
    return jnp.concatenate(rows, axis=0)


def _wcat(p):
    return jnp.concatenate([p["W"], p["root"]], axis=1)


def kernel(x, edges_l6, ev6, edges_l5, ev5, pool_idx, unpool_idx, params):
    P = params
    src6, dst6, evT6, rp6, cnt6 = _prep_level(edges_l6, ev6, N6, N6P, E6P)
    src5, dst5, evT5, rp5, cnt5 = _prep_level(edges_l5, ev5, N5, N5P, E5P)

    g6 = _make_gk(E6P, 4)(_gpar([P["c1"], P["c2"], P["c5"], P["c6"]]), evT6)
    g5 = _make_gk(E5P, 2)(_gpar([P["c3"], P["c4"]]), evT5)
    g1, g2, gc5, gc6 = [g.reshape(-1) for g in g6]
    g3, g4 = [g.reshape(-1) for g in g5]

    agg6 = _make_agg(N6P, NPT6, NPA6, 64, E6P, BE6)
    agg5 = _make_agg(N5P, NPT5, NPA5, 128, E5P, BE5)

    xp = jnp.zeros((N6P, x.shape[1]), f32).at[:N6].set(x)
    b = {k: P[k]["b"].reshape(1, -1) for k in ("c1", "c2", "c3", "c4", "c5", "c6")}

    xw1, r1 = _make_mm(N6P, 32, 192, 64)(xp, _wcat(P["c1"]))
    acc1 = agg6(xw1, g1, src6, dst6, rp6)
    xw2, r2 = _make_mmfin(N6P, 64, 192, 64)(acc1, cnt6, r1, b["c1"], _wcat(P["c2"]))
    acc2 = agg6(xw2, g2, src6, dst6, rp6)
    skip = _make_fin(N6P, 64)(acc2, cnt6, r2, b["c2"])

    pidx = jnp.zeros((N5P * 7,), i32).at[:N5 * 7].set(
        pool_idx.astype(i32).reshape(-1))
    hp = _make_pool()(skip, pidx)

    xw3, r3 = _make_mm(N5P, 64, 384, 128)(hp, _wcat(P["c3"]))
    acc3 = agg5(xw3, g3, src5, dst5, rp5)
    xw4, r4 = _make_mmfin(N5P, 128, 384, 128)(acc3, cnt5, r3, b["c3"], _wcat(P["c4"]))
    acc4 = agg5(xw4, g4, src5, dst5, rp5)
    h4 = _make_fin(N5P, 128)(acc4, cnt5, r4, b["c4"])

    ar5 = jnp.arange(N5, dtype=i32)
    u0 = jnp.concatenate([ar5, unpool_idx[:, 0].astype(i32),
                          jnp.zeros((N6P - N6,), i32)])
    u1 = jnp.concatenate([ar5, unpool_idx[:, 1].astype(i32),
                          jnp.zeros((N6P - N6,), i32)])
    uf = jnp.stack([u0, u1], axis=1).reshape(-1)
    up = _make_unpool()(h4, uf)

    p5 = P["c5"]
    wa = jnp.concatenate([p5["W"][:128], p5["root"][:128]], axis=1)
    wb = jnp.concatenate([p5["W"][128:], p5["root"][128:]], axis=1)
    xw5, r5 = _make_mm2(N6P, 128, 64, 192, 64)(up, skip, wa, wb)
    acc5 = agg6(xw5, gc5, src6, dst6, rp6)
    xw6, r6 = _make_mmfin(N6P, 64, 192, 64)(acc5, cnt6, r5, b["c5"], _wcat(P["c6"]))
    acc6 = agg6(xw6, gc6, src6, dst6, rp6)

    out = _make_final(N6P)(acc6, cnt6, r6, b["c6"], P["fc_w"],
                           P["fc_b"].reshape(1, 2))
    return out[:N6]


# speedup vs baseline: 2.3638x; 1.6134x over previous
import functools

import numpy as np

import jax
import jax.numpy as jnp
from jax import lax
from jax.experimental import pallas as pl
from jax.experimental.pallas import tpu as pltpu
from jax.experimental.pallas import tpu_sc as plsc

N6, N5 = 40962, 10242
E6, E5 = 245760, 61440
N6P, N5P = 41472, 10752
BE = 64

f32 = jnp.float32
i32 = jnp.int32


def _sc_mesh():
    return plsc.VectorSubcoreMesh(core_axis_name="c", subcore_axis_name="s",
                                  num_cores=2, num_subcores=16)


@functools.cache
def _make_agg(Np, cout, E, be):
    h = cout // 2
    kch = 3 * h
    ept = E // 16
    nb = ept // be
    npt_sc = Np // 16
    zr = 48

    @functools.partial(
        pl.kernel, mesh=_sc_mesh(),
        out_type=jax.ShapeDtypeStruct((2, Np, h), f32),
        compiler_params=pltpu.CompilerParams(use_tc_tiling_on_sc=False),
        scratch_types=[
            pltpu.VMEM((be,), i32), pltpu.VMEM((be,), i32),
            pltpu.VMEM((be,), i32), pltpu.VMEM((be,), i32),
            pltpu.VMEM((be,), i32), pltpu.VMEM((be,), i32),
            pltpu.VMEM((3 * be,), f32), pltpu.VMEM((3 * be,), f32),
            pltpu.VMEM((be, kch), f32), pltpu.VMEM((be, kch), f32),
            pltpu.VMEM((be, h), f32), pltpu.VMEM((be, h), f32),
            pltpu.VMEM((zr, h), f32),
            pltpu.VMEM_SHARED((Np, h), f32),
            pltpu.SemaphoreType.DMA, pltpu.SemaphoreType.DMA,
            pltpu.SemaphoreType.DMA, pltpu.SemaphoreType.DMA,
            pltpu.SemaphoreType.DMA, pltpu.SemaphoreType.DMA,
        ],
    )
    def agg(xws, g3, srcs, dsts, out,
            ix0, ix1, dr0, dr1, dr2, dr3, gv0, gv1, rw0, rw1, mg0, mg1,
            zbuf, acc_sh, sm0, sm1, sg0, sg1, ss0, ss1):
        idxb, gvb, rwb, mgb = [ix0, ix1], [gv0, gv1], [rw0, rw1], [mg0, mg1]
        dring = [dr0, dr1, dr2, dr3]
        smb, sgb, ssb = [sm0, sm1], [sg0, sg1], [ss0, ss1]
        c = lax.axis_index("c")
        s = lax.axis_index("s")
        ebase = s * ept
        n0sc = s * npt_sc
        cNp = c * Np

        def zstore(n, _):
            for cc in range(h // 16):
                zbuf[n, pl.ds(cc * 16, 16)] = jnp.zeros((16,), f32)
            return 0
        lax.fori_loop(0, zr, zstore, 0)

        def zcopy(zi, _):
            pltpu.sync_copy(zbuf, acc_sh.at[pl.ds(n0sc + zi * zr, zr)])
            return 0
        lax.fori_loop(0, npt_sc // zr, zcopy, 0)
        plsc.subcore_barrier()

        def issue_meta(bi, par, ring):
            eb = ebase + bi * be
            pltpu.async_copy(srcs.at[pl.ds(eb, be)], idxb[par], smb[par])
            pltpu.async_copy(dsts.at[pl.ds(eb, be)], dring[ring], smb[par])
            for kk in range(3):
                pltpu.async_copy(g3.at[pl.ds(kk * E + eb, be)],
                                 gvb[par].at[pl.ds(kk * be, be)], smb[par])

        def wait_meta(par, ring):
            pltpu.make_async_copy(srcs.at[pl.ds(0, be)], idxb[par],
                                  smb[par]).wait()
            pltpu.make_async_copy(dsts.at[pl.ds(0, be)], dring[ring],
                                  smb[par]).wait()
            for kk in range(3):
                pltpu.make_async_copy(g3.at[pl.ds(0, be)],
                                      gvb[par].at[pl.ds(kk * be, be)],
                                      smb[par]).wait()

        def compute(par):
            def sub(sb, _):
                base = sb * 16
                g0 = gvb[par][pl.ds(base, 16)]
                g1 = gvb[par][pl.ds(be + base, 16)]
                g2 = gvb[par][pl.ds(2 * be + base, 16)]
                rows = rwb[par]
                msg = mgb[par]
                for j2 in range(16):
                    j = base + j2
                    a = g0[j2]
                    b2 = g1[j2]
                    cg = g2[j2]
                    for cc in range(h // 16):
                        o = cc * 16
                        msg[j, pl.ds(o, 16)] = (
                            rows[j, pl.ds(o, 16)] * a
                            + rows[j, pl.ds(h + o, 16)] * b2
                            + rows[j, pl.ds(2 * h + o, 16)] * cg)
                return 0
            lax.fori_loop(0, be // 16, sub, 0)

        issue_meta(0, 0, 0)

        def quad(qi, _):
            for q in range(4):
                b = qi * 4 + q
                par = q % 2
                opar = 1 - par

                @pl.when(b <= nb + 2)
                def _():
                    @pl.when(jnp.logical_and(b >= 1, b <= nb))
                    def _():
                        pltpu.make_async_copy(xws.at[pl.ds(0, be)],
                                              rwb[opar], sgb[opar]).wait()

                    @pl.when(b >= 3)
                    def _():
                        pltpu.make_async_copy(
                            mgb[opar], acc_sh.at[dring[(q + 1) % 4]],
                            ssb[opar]).wait()

                    @pl.when(b < nb)
                    def _():
                        wait_meta(par, q)
                        for kk in range(be // 16):
                            idxb[par][pl.ds(kk * 16, 16)] = (
                                idxb[par][pl.ds(kk * 16, 16)] + cNp)
                        pltpu.async_copy(xws.at[idxb[par]], rwb[par], sgb[par])

                    @pl.when(jnp.logical_and(b >= 1, b <= nb))
                    def _():
                        compute(opar)
                        pltpu.async_copy(mgb[opar],
                                         acc_sh.at[dring[(q + 3) % 4]],
                                         ssb[opar], add=True)

                    @pl.when(b + 1 < nb)
                    def _():
                        issue_meta(b + 1, opar, (q + 1) % 4)
            return 0
        lax.fori_loop(0, (nb + 3 + 3) // 4, quad, 0)

        plsc.subcore_barrier()
        pltpu.sync_copy(acc_sh.at[pl.ds(n0sc, npt_sc)],
                        out.at[c, pl.ds(n0sc, npt_sc)])

    return agg


@functools.cache
def _make_cnt(Np, E, be=128):
    ept = E // 32
    nb = ept // be
    npt_sc = Np // 16
    zr = 48

    @functools.partial(
        pl.kernel, mesh=_sc_mesh(),
        out_type=jax.ShapeDtypeStruct((2, Np, 16), f32),
        compiler_params=pltpu.CompilerParams(use_tc_tiling_on_sc=False),
        scratch_types=[
            pltpu.VMEM((be,), i32), pltpu.VMEM((be,), i32),
            pltpu.VMEM((be,), i32), pltpu.VMEM((be,), i32),
            pltpu.VMEM((be, 16), f32),
            pltpu.VMEM((zr, 16), f32),
            pltpu.VMEM_SHARED((Np, 16), f32),
            pltpu.SemaphoreType.DMA, pltpu.SemaphoreType.DMA,
            pltpu.SemaphoreType.DMA, pltpu.SemaphoreType.DMA,
        ],
    )
    def cntk(dsts, out, dr0, dr1, dr2, dr3, ones_v, zbuf, acc_sh,
             sm0, sm1, ss0, ss1):
        dring = [dr0, dr1, dr2, dr3]
        smb, ssb = [sm0, sm1], [ss0, ss1]
        c = lax.axis_index("c")
        s = lax.axis_index("s")
        wid = s * 2 + c
        ebase = wid * ept
        n0sc = s * npt_sc

        def fill(n, _):
            ones_v[n, pl.ds(0, 16)] = jnp.full((16,), 1.0, f32)
            for cc in range(1):
                pass
            zbuf[jnp.minimum(n, zr - 1), pl.ds(0, 16)] = jnp.zeros((16,), f32)
            return 0
        lax.fori_loop(0, be, fill, 0)

        def zcopy(zi, _):
            pltpu.sync_copy(zbuf, acc_sh.at[pl.ds(n0sc + zi * zr, zr)])
            return 0
        lax.fori_loop(0, npt_sc // zr, zcopy, 0)
        plsc.subcore_barrier()

        def issue_meta(bi, par, ring):
            pltpu.async_copy(dsts.at[pl.ds(ebase + bi * be, be)],
                             dring[ring], smb[par])

        issue_meta(0, 0, 0)

        def quad(qi, _):
            for q in range(4):
                b = qi * 4 + q
                par = q % 2

                @pl.when(b <= nb + 1)
                def _():
                    @pl.when(jnp.logical_and(b >= 2, b <= nb + 1))
                    def _():
                        pltpu.make_async_copy(ones_v,
                                              acc_sh.at[dring[(q + 2) % 4]],
                                              ssb[par]).wait()

                    @pl.when(b < nb)
                    def _():
                        pltpu.make_async_copy(dsts.at[pl.ds(0, be)],
                                              dring[q], smb[par]).wait()
                        pltpu.async_copy(ones_v, acc_sh.at[dring[q]],
                                         ssb[par], add=True)

                    @pl.when(b + 1 < nb)
                    def _():
                        issue_meta(b + 1, 1 - par, (q + 1) % 4)
            return 0
        lax.fori_loop(0, (nb + 2 + 3) // 4, quad, 0)

        plsc.subcore_barrier()
        pltpu.sync_copy(acc_sh.at[pl.ds(n0sc, npt_sc)],
                        out.at[c, pl.ds(n0sc, npt_sc)])

    return cntk


@functools.cache
def _make_pool():
    npt, C = N5P // 32, 64
    nbn = 8

    @functools.partial(
        pl.kernel, mesh=_sc_mesh(),
        out_type=jax.ShapeDtypeStruct((N5P, C), f32),
        compiler_params=pltpu.CompilerParams(use_tc_tiling_on_sc=False),
        scratch_types=[
            pltpu.VMEM((7 * nbn,), i32), pltpu.VMEM((7 * nbn,), i32),
            pltpu.VMEM((7 * nbn, C), f32), pltpu.VMEM((7 * nbn, C), f32),
            pltpu.VMEM((npt, C), f32),
            pltpu.SemaphoreType.DMA, pltpu.SemaphoreType.DMA,
            pltpu.SemaphoreType.DMA, pltpu.SemaphoreType.DMA,
        ],
    )
    def pool(skip, pidx, out, ix0, ix1, rw0, rw1, out_v, si0, si1, sg0, sg1):
        idxb, rwb = [ix0, ix1], [rw0, rw1]
        sib, sgb = [si0, si1], [sg0, sg1]
        wid = lax.axis_index("s") * 2 + lax.axis_index("c")
        n0 = wid * npt
        nb = npt // nbn

        def issue_idx(bi, par):
            pltpu.async_copy(pidx.at[pl.ds((n0 + bi * nbn) * 7, 7 * nbn)],
                             idxb[par], sib[par])

        issue_idx(0, 0)

        def pair(i, _):
            for par in range(2):
                b = i * 2 + par
                opar = 1 - par

                @pl.when(b <= nb)
                def _():
                    @pl.when(b >= 1)
                    def _():
                        pltpu.make_async_copy(skip.at[pl.ds(0, 7 * nbn)],
                                              rwb[opar], sgb[opar]).wait()

                    @pl.when(b < nb)
                    def _():
                        pltpu.make_async_copy(pidx.at[pl.ds(0, 7 * nbn)],
                                              idxb[par], sib[par]).wait()
                        pltpu.async_copy(skip.at[idxb[par]], rwb[par], sgb[par])

                    @pl.when(b + 1 < nb)
                    def _():
                        issue_idx(b + 1, opar)

                    @pl.when(b >= 1)
                    def _():
                        bm = b - 1
                        rows = rwb[opar]
                        for j in range(nbn):
                            for cc in range(C // 16):
                                o = cc * 16
                                v = rows[7 * j, pl.ds(o, 16)]
                                for t in range(1, 7):
                                    v = jnp.maximum(v, rows[7 * j + t,
                                                            pl.ds(o, 16)])
                                out_v[bm * nbn + j, pl.ds(o, 16)] = v
            return 0
        lax.fori_loop(0, (nb + 2) // 2, pair, 0)
        pltpu.sync_copy(out_v, out.at[pl.ds(n0, npt)])

    return pool


@functools.cache
def _make_unpool():
    npt, C = N6P // 32, 128
    nbr = 48

    @functools.partial(
        pl.kernel, mesh=_sc_mesh(),
        out_type=jax.ShapeDtypeStruct((N6P, C), f32),
        compiler_params=pltpu.CompilerParams(use_tc_tiling_on_sc=False),
        scratch_types=[
            pltpu.VMEM((2 * nbr,), i32), pltpu.VMEM((2 * nbr,), i32),
            pltpu.VMEM((2 * nbr, C), f32), pltpu.VMEM((2 * nbr, C), f32),
            pltpu.VMEM((nbr, C), f32), pltpu.VMEM((nbr, C), f32),
            pltpu.SemaphoreType.DMA, pltpu.SemaphoreType.DMA,
            pltpu.SemaphoreType.DMA, pltpu.SemaphoreType.DMA,
            pltpu.SemaphoreType.DMA, pltpu.SemaphoreType.DMA,
        ],
    )
    def unpool(h4, uf, out, ix0, ix1, rw0, rw1, ov0, ov1,
               si0, si1, sg0, sg1, so0, so1):
        idxb, rwb, ovb = [ix0, ix1], [rw0, rw1], [ov0, ov1]
        sib, sgb, sob = [si0, si1], [sg0, sg1], [so0, so1]
        wid = lax.axis_index("s") * 2 + lax.axis_index("c")
        n0 = wid * npt
        nb = npt // nbr

        def issue_idx(bi, par):
            pltpu.async_copy(uf.at[pl.ds((n0 + bi * nbr) * 2, 2 * nbr)],
                             idxb[par], sib[par])

        issue_idx(0, 0)

        def pair(i, _):
            for par in range(2):
                b = i * 2 + par
                opar = 1 - par

                @pl.when(b <= nb + 1)
                def _():
                    @pl.when(jnp.logical_and(b >= 1, b <= nb))
                    def _():
                        pltpu.make_async_copy(h4.at[pl.ds(0, 2 * nbr)],
                                              rwb[opar], sgb[opar]).wait()

                    @pl.when(jnp.logical_and(b >= 2, b <= nb + 1))
                    def _():
                        pltpu.make_async_copy(ovb[par],
                                              out.at[pl.ds(0, nbr)],
                                              sob[par]).wait()

                    @pl.when(b < nb)
                    def _():
                        pltpu.make_async_copy(uf.at[pl.ds(0, 2 * nbr)],
                                              idxb[par], sib[par]).wait()
                        pltpu.async_copy(h4.at[idxb[par]], rwb[par], sgb[par])

                    @pl.when(b + 1 < nb)
                    def _():
                        issue_idx(b + 1, opar)

                    @pl.when(jnp.logical_and(b >= 1, b <= nb))
                    def _():
                        bm = b - 1
                        rows = rwb[opar]

                        def row(j, _):
                            for cc in range(C // 16):
                                o = cc * 16
                                ovb[opar][j, pl.ds(o, 16)] = (
                                    rows[2 * j, pl.ds(o, 16)]
                                    + rows[2 * j + 1, pl.ds(o, 16)]) * 0.5
                            return 0
                        lax.fori_loop(0, nbr, row, 0)
                        pltpu.async_copy(ovb[opar],
                                         out.at[pl.ds(n0 + bm * nbr, nbr)],
                                         sob[opar])
            return 0
        lax.fori_loop(0, (nb + 3) // 2, pair, 0)

    return unpool


@functools.cache
def _make_gk(E, nconv, blk=2048):
    def body(par_ref, ev_ref, *out_refs):
        e0 = ev_ref[0:1, :]
        e1 = ev_ref[1:2, :]
        for ic in range(nconv):
            for k in range(3):
                a0 = par_ref[ic * 3 + k, 0]
                a1 = par_ref[ic * 3 + k, 1]
                d0 = e0 - par_ref[ic * 3 + k, 2]
                d1 = e1 - par_ref[ic * 3 + k, 3]
                out_refs[ic][k:k + 1, :] = jnp.exp(a0 * d0 * d0 + a1 * d1 * d1)

    return pl.pallas_call(
        body,
        grid=(pl.cdiv(E, blk),),
        in_specs=[pl.BlockSpec(memory_space=pltpu.SMEM),
                  pl.BlockSpec((2, blk), lambda i: (0, i))],
        out_specs=[pl.BlockSpec((3, blk), lambda i: (0, i))] * nconv,
        out_shape=[jax.ShapeDtypeStruct((3, E), f32)] * nconv,
    )


@functools.cache
def _make_mm(Np, cin, cout, blk=256):
    kch = 3 * (cout // 2)

    def body(x_ref, w_ref, xw_ref, r_ref):
        y = jnp.dot(x_ref[...], w_ref[...], preferred_element_type=f32)
        xw_ref[0] = y[:, :kch]
        xw_ref[1] = y[:, kch:2 * kch]
        r_ref[...] = y[:, 2 * kch:]

    return pl.pallas_call(
        body,
        grid=(Np // blk,),
        in_specs=[pl.BlockSpec((blk, cin), lambda i: (i, 0)),
                  pl.BlockSpec((cin, 4 * cout), lambda i: (0, 0))],
        out_specs=[pl.BlockSpec((2, blk, kch), lambda i: (0, i, 0)),
                   pl.BlockSpec((blk, cout), lambda i: (i, 0))],
        out_shape=[jax.ShapeDtypeStruct((2, Np, kch), f32),
                   jax.ShapeDtypeStruct((Np, cout), f32)],
    )


@functools.cache
def _make_mmfin(Np, cp, cout, blk=256):
    hp = cp // 2
    kch = 3 * (cout // 2)

    def body(a0_ref, a1_ref, cnt_ref, r_ref, b_ref, w_ref, xw_ref, r2_ref):
        acc = jnp.concatenate([a0_ref[...], a1_ref[...]], axis=1)
        inv = 1.0 / jnp.maximum(cnt_ref[...], 1.0)
        hh = jnp.maximum(acc * inv + r_ref[...] + b_ref[...], 0.0)
        y = jnp.dot(hh, w_ref[...], preferred_element_type=f32)
        xw_ref[0] = y[:, :kch]
        xw_ref[1] = y[:, kch:2 * kch]
        r2_ref[...] = y[:, 2 * kch:]

    return pl.pallas_call(
        body,
        grid=(Np // blk,),
        in_specs=[pl.BlockSpec((blk, hp), lambda i: (i, 0)),
                  pl.BlockSpec((blk, hp), lambda i: (i, 0)),
                  pl.BlockSpec((blk, 1), lambda i: (i, 0)),
                  pl.BlockSpec((blk, cp), lambda i: (i, 0)),
                  pl.BlockSpec((1, cp), lambda i: (0, 0)),
                  pl.BlockSpec((cp, 4 * cout), lambda i: (0, 0))],
        out_specs=[pl.BlockSpec((2, blk, kch), lambda i: (0, i, 0)),
                   pl.BlockSpec((blk, cout), lambda i: (i, 0))],
        out_shape=[jax.ShapeDtypeStruct((2, Np, kch), f32),
                   jax.ShapeDtypeStruct((Np, cout), f32)],
    )


@functools.cache
def _make_fin(Np, cp, blk=256):
    hp = cp // 2

    def body(a0_ref, a1_ref, cnt_ref, r_ref, b_ref, h_ref):
        acc = jnp.concatenate([a0_ref[...], a1_ref[...]], axis=1)
        inv = 1.0 / jnp.maximum(cnt_ref[...], 1.0)
        h_ref[...] = jnp.maximum(acc * inv + r_ref[...] + b_ref[...], 0.0)

    return pl.pallas_call(
        body,
        grid=(Np // blk,),
        in_specs=[pl.BlockSpec((blk, hp), lambda i: (i, 0)),
                  pl.BlockSpec((blk, hp), lambda i: (i, 0)),
                  pl.BlockSpec((blk, 1), lambda i: (i, 0)),
                  pl.BlockSpec((blk, cp), lambda i: (i, 0)),
                  pl.BlockSpec((1, cp), lambda i: (0, 0))],
        out_specs=pl.BlockSpec((blk, cp), lambda i: (i, 0)),
        out_shape=jax.ShapeDtypeStruct((Np, cp), f32),
    )


@functools.cache
def _make_mm2(Np, c1, c2, cout, blk=256):
    kch = 3 * (cout // 2)

    def body(a_ref, b_ref, wa_ref, wb_ref, xw_ref, r_ref):
        y = (jnp.dot(a_ref[...], wa_ref[...], preferred_element_type=f32)
             + jnp.dot(b_ref[...], wb_ref[...], preferred_element_type=f32))
        xw_ref[0] = y[:, :kch]
        xw_ref[1] = y[:, kch:2 * kch]
        r_ref[...] = y[:, 2 * kch:]

    return pl.pallas_call(
        body,
        grid=(Np // blk,),
        in_specs=[pl.BlockSpec((blk, c1), lambda i: (i, 0)),
                  pl.BlockSpec((blk, c2), lambda i: (i, 0)),
                  pl.BlockSpec((c1, 4 * cout), lambda i: (0, 0)),
                  pl.BlockSpec((c2, 4 * cout), lambda i: (0, 0))],
        out_specs=[pl.BlockSpec((2, blk, kch), lambda i: (0, i, 0)),
                   pl.BlockSpec((blk, cout), lambda i: (i, 0))],
        out_shape=[jax.ShapeDtypeStruct((2, Np, kch), f32),
                   jax.ShapeDtypeStruct((Np, cout), f32)],
    )


@functools.cache
def _make_final(Np, cp=64, blk=256):
    hp = cp // 2

    def body(a0_ref, a1_ref, cnt_ref, r_ref, b_ref, fcw_ref, fcb_ref, o_ref):
        acc = jnp.concatenate([a0_ref[...], a1_ref[...]], axis=1)
        inv = 1.0 / jnp.maximum(cnt_ref[...], 1.0)
        hh = jnp.maximum(acc * inv + r_ref[...] + b_ref[...], 0.0)
        lg = jnp.dot(hh, fcw_ref[...], preferred_element_type=f32) + fcb_ref[...]
        mx = jnp.max(lg, axis=1, keepdims=True)
        lse = mx + jnp.log(jnp.sum(jnp.exp(lg - mx), axis=1, keepdims=True))
        o_ref[...] = lg - lse

    return pl.pallas_call(
        body,
        grid=(Np // blk,),
        in_specs=[pl.BlockSpec((blk, hp), lambda i: (i, 0)),
                  pl.BlockSpec((blk, hp), lambda i: (i, 0)),
                  pl.BlockSpec((blk, 1), lambda i: (i, 0)),
                  pl.BlockSpec((blk, cp), lambda i: (i, 0)),
                  pl.BlockSpec((1, cp), lambda i: (0, 0)),
                  pl.BlockSpec((cp, 2), lambda i: (0, 0)),
                  pl.BlockSpec((1, 2), lambda i: (0, 0))],
        out_specs=pl.BlockSpec((blk, 2), lambda i: (i, 0)),
        out_shape=jax.ShapeDtypeStruct((Np, 2), f32),
    )


def _gpar(ps):
    rows = []
    for p in ps:
        a = -0.5 / (p["sigma"] ** 2 + 1e-8)
        rows.append(jnp.concatenate([a, p["mu"]], axis=1))
    return jnp.concatenate(rows, axis=0)


def _wperm(W, root, cout):
    h = cout // 2
    order = np.array([k * cout + c * h + j
                      for c in range(2) for k in range(3) for j in range(h)])
    return jnp.concatenate([W[:, order], root], axis=1)


def kernel(x, edges_l6, ev6, edges_l5, ev5, pool_idx, unpool_idx, params):
    P = params
    src6 = edges_l6[0].astype(i32)
    dst6 = edges_l6[1].astype(i32)
    src5 = edges_l5[0].astype(i32)
    dst5 = edges_l5[1].astype(i32)

    cnt6r = _make_cnt(N6P, E6)(dst6)
    cnt6 = cnt6r[0, :, :1] + cnt6r[1, :, :1]
    cnt5r = _make_cnt(N5P, E5)(dst5)
    cnt5 = cnt5r[0, :, :1] + cnt5r[1, :, :1]

    g6 = _make_gk(E6, 4)(_gpar([P["c1"], P["c2"], P["c5"], P["c6"]]), ev6.T)
    g5 = _make_gk(E5, 2)(_gpar([P["c3"], P["c4"]]), ev5.T)
    g1, g2, gc5, gc6 = [g.reshape(-1) for g in g6]
    g3, g4 = [g.reshape(-1) for g in g5]

    agg6 = _make_agg(N6P, 64, E6, BE)
    agg5 = _make_agg(N5P, 128, E5, BE)

    xp = jnp.zeros((N6P, x.shape[1]), f32).at[:N6].set(x)
    b = {k: P[k]["b"].reshape(1, -1) for k in ("c1", "c2", "c3", "c4", "c5", "c6")}

    xw1, r1 = _make_mm(N6P, 32, 64)(xp, _wperm(P["c1"]["W"], P["c1"]["root"], 64))
    a1 = agg6(xw1.reshape(2 * N6P, -1), g1, src6, dst6)
    xw2, r2 = _make_mmfin(N6P, 64, 64)(a1[0], a1[1], cnt6, r1, b["c1"],
                                       _wperm(P["c2"]["W"], P["c2"]["root"], 64))
    a2 = agg6(xw2.reshape(2 * N6P, -1), g2, src6, dst6)
    skip = _make_fin(N6P, 64)(a2[0], a2[1], cnt6, r2, b["c2"])

    pidx = jnp.zeros((N5P * 7,), i32).at[:N5 * 7].set(
        pool_idx.astype(i32).reshape(-1))
    hp = _make_pool()(skip, pidx)

    xw3, r3 = _make_mm(N5P, 64, 128)(hp, _wperm(P["c3"]["W"], P["c3"]["root"], 128))
    a3 = agg5(xw3.reshape(2 * N5P, -1), g3, src5, dst5)
    xw4, r4 = _make_mmfin(N5P, 128, 128)(a3[0], a3[1], cnt5, r3, b["c3"],
                                         _wperm(P["c4"]["W"], P["c4"]["root"], 128))
    a4 = agg5(xw4.reshape(2 * N5P, -1), g4, src5, dst5)
    h4 = _make_fin(N5P, 128)(a4[0], a4[1], cnt5, r4, b["c4"])

    ar5 = jnp.arange(N5, dtype=i32)
    u0 = jnp.concatenate([ar5, unpool_idx[:, 0].astype(i32),
                          jnp.zeros((N6P - N6,), i32)])
    u1 = jnp.concatenate([ar5, unpool_idx[:, 1].astype(i32),
                          jnp.zeros((N6P - N6,), i32)])
    uf = jnp.stack([u0, u1], axis=1).reshape(-1)
    up = _make_unpool()(h4, uf)

    p5 = P["c5"]
    wa = _wperm(p5["W"][:128], p5["root"][:128], 64)
    wb = _wperm(p5["W"][128:], p5["root"][128:], 64)
    xw5, r5 = _make_mm2(N6P, 128, 64, 64)(up, skip, wa, wb)
    a5 = agg6(xw5.reshape(2 * N6P, -1), gc5, src6, dst6)
    xw6, r6 = _make_mmfin(N6P, 64, 64)(a5[0], a5[1], cnt6, r5, b["c5"],
                                       _wperm(P["c6"]["W"], P["c6"]["root"], 64))
    a6 = agg6(xw6.reshape(2 * N6P, -1), gc6, src6, dst6)

    out = _make_final(N6P)(a6[0], a6[1], cnt6, r6, b["c6"], P["fc_w"],
                           P["fc_b"].reshape(1, 2))
    return out[:N6]


# speedup vs baseline: 2.7521x; 1.1643x over previous
import functools

import numpy as np

import jax
import jax.numpy as jnp
from jax import lax
from jax.experimental import pallas as pl
from jax.experimental.pallas import tpu as pltpu
from jax.experimental.pallas import tpu_sc as plsc

N6, N5 = 40962, 10242
E6, E5 = 245760, 61440
N6P, N5P = 41472, 10752
BE = 64

f32 = jnp.float32
i32 = jnp.int32


def _sc_mesh():
    return plsc.VectorSubcoreMesh(core_axis_name="c", subcore_axis_name="s",
                                  num_cores=2, num_subcores=16)


@functools.cache
def _make_agg(Np, cout, E, be):
    h = cout // 2
    kch = 3 * h
    ept = E // 16
    nb = ept // be
    npt_sc = Np // 16
    zr = 48

    @functools.partial(
        pl.kernel, mesh=_sc_mesh(),
        out_type=jax.ShapeDtypeStruct((2, Np, h), f32),
        compiler_params=pltpu.CompilerParams(use_tc_tiling_on_sc=False),
        scratch_types=[
            pltpu.VMEM((be,), i32), pltpu.VMEM((be,), i32),
            pltpu.VMEM((be,), i32), pltpu.VMEM((be,), i32),
            pltpu.VMEM((be,), i32), pltpu.VMEM((be,), i32),
            pltpu.VMEM((3 * be,), f32), pltpu.VMEM((3 * be,), f32),
            pltpu.VMEM((3 * be,), f32), pltpu.VMEM((3 * be,), f32),
            pltpu.VMEM((be, kch), f32), pltpu.VMEM((be, kch), f32),
            pltpu.VMEM((be, h), f32), pltpu.VMEM((be, h), f32),
            pltpu.VMEM((zr, h), f32),
            pltpu.VMEM_SHARED((Np, h), f32),
            pltpu.SemaphoreType.DMA, pltpu.SemaphoreType.DMA,
            pltpu.SemaphoreType.DMA, pltpu.SemaphoreType.DMA,
            pltpu.SemaphoreType.DMA, pltpu.SemaphoreType.DMA,
        ],
    )
    def agg(xws, g3, srcs, dsts, out,
            ix0, ix1, dr0, dr1, dr2, dr3, gv0, gv1, gv2, gv3,
            rw0, rw1, mg0, mg1,
            zbuf, acc_sh, sm0, sm1, sg0, sg1, ss0, ss1):
        idxb, rwb, mgb = [ix0, ix1], [rw0, rw1], [mg0, mg1]
        gring = [gv0, gv1, gv2, gv3]
        dring = [dr0, dr1, dr2, dr3]
        smb, sgb, ssb = [sm0, sm1], [sg0, sg1], [ss0, ss1]
        c = lax.axis_index("c")
        s = lax.axis_index("s")
        ebase = s * ept
        n0sc = s * npt_sc
        cNp = c * Np

        def zstore(n, _):
            for cc in range(h // 16):
                zbuf[n, pl.ds(cc * 16, 16)] = jnp.zeros((16,), f32)
            return 0
        lax.fori_loop(0, zr, zstore, 0)

        def zcopy(zi, _):
            pltpu.sync_copy(zbuf, acc_sh.at[pl.ds(n0sc + zi * zr, zr)])
            return 0
        lax.fori_loop(0, npt_sc // zr, zcopy, 0)
        plsc.subcore_barrier()

        def issue_meta(bi, par, ring):
            eb = ebase + bi * be
            pltpu.async_copy(srcs.at[pl.ds(eb, be)], idxb[par], smb[par])
            pltpu.async_copy(dsts.at[pl.ds(eb, be)], dring[ring], smb[par])
            for kk in range(3):
                pltpu.async_copy(g3.at[pl.ds(kk * E + eb, be)],
                                 gring[ring].at[pl.ds(kk * be, be)], smb[par])

        def wait_meta(par, ring):
            pltpu.make_async_copy(srcs.at[pl.ds(0, be)], idxb[par],
                                  smb[par]).wait()
            pltpu.make_async_copy(dsts.at[pl.ds(0, be)], dring[ring],
                                  smb[par]).wait()
            for kk in range(3):
                pltpu.make_async_copy(g3.at[pl.ds(0, be)],
                                      gring[ring].at[pl.ds(kk * be, be)],
                                      smb[par]).wait()

        def compute(par, ring):
            def sub(sb, _):
                base = sb * 16
                g0 = gring[ring][pl.ds(base, 16)]
                g1 = gring[ring][pl.ds(be + base, 16)]
                g2 = gring[ring][pl.ds(2 * be + base, 16)]
                rows = rwb[par]
                msg = mgb[par]
                for j2 in range(16):
                    j = base + j2
                    a = g0[j2]
                    b2 = g1[j2]
                    cg = g2[j2]
                    for cc in range(h // 16):
                        o = cc * 16
                        msg[j, pl.ds(o, 16)] = (
                            rows[j, pl.ds(o, 16)] * a
                            + rows[j, pl.ds(h + o, 16)] * b2
                            + rows[j, pl.ds(2 * h + o, 16)] * cg)
                return 0
            lax.fori_loop(0, be // 16, sub, 0)

        issue_meta(0, 0, 0)

        def quad(qi, _):
            for q in range(4):
                b = qi * 4 + q
                par = q % 2
                opar = 1 - par

                @pl.when(b <= nb + 2)
                def _():
                    @pl.when(b >= 3)
                    def _():
                        pltpu.make_async_copy(
                            mgb[opar], acc_sh.at[dring[(q + 1) % 4]],
                            ssb[opar]).wait()

                    @pl.when(b < nb)
                    def _():
                        wait_meta(par, q)
                        for kk in range(be // 16):
                            idxb[par][pl.ds(kk * 16, 16)] = (
                                idxb[par][pl.ds(kk * 16, 16)] + cNp)
                        pltpu.async_copy(xws.at[idxb[par]], rwb[par], sgb[par])

                    @pl.when(jnp.logical_and(b >= 1, b <= nb))
                    def _():
                        pltpu.make_async_copy(xws.at[pl.ds(0, be)],
                                              rwb[opar], sgb[opar]).wait()

                    @pl.when(b + 1 < nb)
                    def _():
                        issue_meta(b + 1, opar, (q + 1) % 4)

                    @pl.when(jnp.logical_and(b >= 1, b <= nb))
                    def _():
                        compute(opar, (q + 3) % 4)
                        pltpu.async_copy(mgb[opar],
                                         acc_sh.at[dring[(q + 3) % 4]],
                                         ssb[opar], add=True)
            return 0
        lax.fori_loop(0, (nb + 3 + 3) // 4, quad, 0)

        plsc.subcore_barrier()
        pltpu.sync_copy(acc_sh.at[pl.ds(n0sc, npt_sc)],
                        out.at[c, pl.ds(n0sc, npt_sc)])

    return agg


@functools.cache
def _make_cnt(Np, E, be=128):
    ept = E // 32
    nb = ept // be
    npt_sc = Np // 16
    zr = 48

    @functools.partial(
        pl.kernel, mesh=_sc_mesh(),
        out_type=jax.ShapeDtypeStruct((2, Np, 16), f32),
        compiler_params=pltpu.CompilerParams(use_tc_tiling_on_sc=False),
        scratch_types=[
            pltpu.VMEM((be,), i32), pltpu.VMEM((be,), i32),
            pltpu.VMEM((be,), i32), pltpu.VMEM((be,), i32),
            pltpu.VMEM((be, 16), f32),
            pltpu.VMEM((zr, 16), f32),
            pltpu.VMEM_SHARED((Np, 16), f32),
            pltpu.SemaphoreType.DMA, pltpu.SemaphoreType.DMA,
            pltpu.SemaphoreType.DMA, pltpu.SemaphoreType.DMA,
        ],
    )
    def cntk(dsts, out, dr0, dr1, dr2, dr3, ones_v, zbuf, acc_sh,
             sm0, sm1, ss0, ss1):
        dring = [dr0, dr1, dr2, dr3]
        smb, ssb = [sm0, sm1], [ss0, ss1]
        c = lax.axis_index("c")
        s = lax.axis_index("s")
        wid = s * 2 + c
        ebase = wid * ept
        n0sc = s * npt_sc

        def fill(n, _):
            ones_v[n, pl.ds(0, 16)] = jnp.full((16,), 1.0, f32)
            for cc in range(1):
                pass
            zbuf[jnp.minimum(n, zr - 1), pl.ds(0, 16)] = jnp.zeros((16,), f32)
            return 0
        lax.fori_loop(0, be, fill, 0)

        def zcopy(zi, _):
            pltpu.sync_copy(zbuf, acc_sh.at[pl.ds(n0sc + zi * zr, zr)])
            return 0
        lax.fori_loop(0, npt_sc // zr, zcopy, 0)
        plsc.subcore_barrier()

        def issue_meta(bi, par, ring):
            pltpu.async_copy(dsts.at[pl.ds(ebase + bi * be, be)],
                             dring[ring], smb[par])

        issue_meta(0, 0, 0)

        def quad(qi, _):
            for q in range(4):
                b = qi * 4 + q
                par = q % 2

                @pl.when(b <= nb + 1)
                def _():
                    @pl.when(jnp.logical_and(b >= 2, b <= nb + 1))
                    def _():
                        pltpu.make_async_copy(ones_v,
                                              acc_sh.at[dring[(q + 2) % 4]],
                                              ssb[par]).wait()

                    @pl.when(b < nb)
                    def _():
                        pltpu.make_async_copy(dsts.at[pl.ds(0, be)],
                                              dring[q], smb[par]).wait()
                        pltpu.async_copy(ones_v, acc_sh.at[dring[q]],
                                         ssb[par], add=True)

                    @pl.when(b + 1 < nb)
                    def _():
                        issue_meta(b + 1, 1 - par, (q + 1) % 4)
            return 0
        lax.fori_loop(0, (nb + 2 + 3) // 4, quad, 0)

        plsc.subcore_barrier()
        pltpu.sync_copy(acc_sh.at[pl.ds(n0sc, npt_sc)],
                        out.at[c, pl.ds(n0sc, npt_sc)])

    return cntk


@functools.cache
def _make_pool():
    npt, C = N5P // 32, 64
    nbn = 16

    @functools.partial(
        pl.kernel, mesh=_sc_mesh(),
        out_type=jax.ShapeDtypeStruct((N5P, C), f32),
        compiler_params=pltpu.CompilerParams(use_tc_tiling_on_sc=False),
        scratch_types=[
            pltpu.VMEM((7 * nbn,), i32), pltpu.VMEM((7 * nbn,), i32),
            pltpu.VMEM((7 * nbn, C), f32), pltpu.VMEM((7 * nbn, C), f32),
            pltpu.VMEM((npt, C), f32),
            pltpu.SemaphoreType.DMA, pltpu.SemaphoreType.DMA,
            pltpu.SemaphoreType.DMA, pltpu.SemaphoreType.DMA,
        ],
    )
    def pool(skip, pidx, out, ix0, ix1, rw0, rw1, out_v, si0, si1, sg0, sg1):
        idxb, rwb = [ix0, ix1], [rw0, rw1]
        sib, sgb = [si0, si1], [sg0, sg1]
        wid = lax.axis_index("s") * 2 + lax.axis_index("c")
        n0 = wid * npt
        nb = npt // nbn

        def issue_idx(bi, par):
            pltpu.async_copy(pidx.at[pl.ds((n0 + bi * nbn) * 7, 7 * nbn)],
                             idxb[par], sib[par])

        issue_idx(0, 0)

        def pair(i, _):
            for par in range(2):
                b = i * 2 + par
                opar = 1 - par

                @pl.when(b <= nb)
                def _():
                    @pl.when(b < nb)
                    def _():
                        pltpu.make_async_copy(pidx.at[pl.ds(0, 7 * nbn)],
                                              idxb[par], sib[par]).wait()
                        pltpu.async_copy(skip.at[idxb[par]], rwb[par], sgb[par])

                    @pl.when(b >= 1)
                    def _():
                        pltpu.make_async_copy(skip.at[pl.ds(0, 7 * nbn)],
                                              rwb[opar], sgb[opar]).wait()

                    @pl.when(b + 1 < nb)
                    def _():
                        issue_idx(b + 1, opar)

                    @pl.when(b >= 1)
                    def _():
                        bm = b - 1
                        rows = rwb[opar]
                        for j in range(nbn):
                            for cc in range(C // 16):
                                o = cc * 16
                                v = rows[7 * j, pl.ds(o, 16)]
                                for t in range(1, 7):
                                    v = jnp.maximum(v, rows[7 * j + t,
                                                            pl.ds(o, 16)])
                                out_v[bm * nbn + j, pl.ds(o, 16)] = v
            return 0
        lax.fori_loop(0, (nb + 2) // 2, pair, 0)
        pltpu.sync_copy(out_v, out.at[pl.ds(n0, npt)])

    return pool


@functools.cache
def _make_unpool():
    npt, C = N6P // 32, 128
    nbr = 48

    @functools.partial(
        pl.kernel, mesh=_sc_mesh(),
        out_type=jax.ShapeDtypeStruct((N6P, C), f32),
        compiler_params=pltpu.CompilerParams(use_tc_tiling_on_sc=False),
        scratch_types=[
            pltpu.VMEM((2 * nbr,), i32), pltpu.VMEM((2 * nbr,), i32),
            pltpu.VMEM((2 * nbr, C), f32), pltpu.VMEM((2 * nbr, C), f32),
            pltpu.VMEM((nbr, C), f32), pltpu.VMEM((nbr, C), f32),
            pltpu.SemaphoreType.DMA, pltpu.SemaphoreType.DMA,
            pltpu.SemaphoreType.DMA, pltpu.SemaphoreType.DMA,
            pltpu.SemaphoreType.DMA, pltpu.SemaphoreType.DMA,
        ],
    )
    def unpool(h4, uf, out, ix0, ix1, rw0, rw1, ov0, ov1,
               si0, si1, sg0, sg1, so0, so1):
        idxb, rwb, ovb = [ix0, ix1], [rw0, rw1], [ov0, ov1]
        sib, sgb, sob = [si0, si1], [sg0, sg1], [so0, so1]
        wid = lax.axis_index("s") * 2 + lax.axis_index("c")
        n0 = wid * npt
        nb = npt // nbr

        def issue_idx(bi, par):
            pltpu.async_copy(uf.at[pl.ds((n0 + bi * nbr) * 2, 2 * nbr)],
                             idxb[par], sib[par])

        issue_idx(0, 0)

        def pair(i, _):
            for par in range(2):
                b = i * 2 + par
                opar = 1 - par

                @pl.when(b <= nb + 1)
                def _():
                    @pl.when(jnp.logical_and(b >= 2, b <= nb + 1))
                    def _():
                        pltpu.make_async_copy(ovb[par],
                                              out.at[pl.ds(0, nbr)],
                                              sob[par]).wait()

                    @pl.when(b < nb)
                    def _():
                        pltpu.make_async_copy(uf.at[pl.ds(0, 2 * nbr)],
                                              idxb[par], sib[par]).wait()
                        pltpu.async_copy(h4.at[idxb[par]], rwb[par], sgb[par])

                    @pl.when(jnp.logical_and(b >= 1, b <= nb))
                    def _():
                        pltpu.make_async_copy(h4.at[pl.ds(0, 2 * nbr)],
                                              rwb[opar], sgb[opar]).wait()

                    @pl.when(b + 1 < nb)
                    def _():
                        issue_idx(b + 1, opar)

                    @pl.when(jnp.logical_and(b >= 1, b <= nb))
                    def _():
                        bm = b - 1
                        rows = rwb[opar]

                        def row(j, _):
                            for cc in range(C // 16):
                                o = cc * 16
                                ovb[opar][j, pl.ds(o, 16)] = (
                                    rows[2 * j, pl.ds(o, 16)]
                                    + rows[2 * j + 1, pl.ds(o, 16)]) * 0.5
                            return 0
                        lax.fori_loop(0, nbr, row, 0)
                        pltpu.async_copy(ovb[opar],
                                         out.at[pl.ds(n0 + bm * nbr, nbr)],
                                         sob[opar])
            return 0
        lax.fori_loop(0, (nb + 3) // 2, pair, 0)

    return unpool


@functools.cache
def _make_gk(E, nconv, blk=2048):
    def body(par_ref, ev_ref, *out_refs):
        e0 = ev_ref[0:1, :]
        e1 = ev_ref[1:2, :]
        for ic in range(nconv):
            for k in range(3):
                a0 = par_ref[ic * 3 + k, 0]
                a1 = par_ref[ic * 3 + k, 1]
                d0 = e0 - par_ref[ic * 3 + k, 2]
                d1 = e1 - par_ref[ic * 3 + k, 3]
                out_refs[ic][k:k + 1, :] = jnp.exp(a0 * d0 * d0 + a1 * d1 * d1)

    return pl.pallas_call(
        body,
        grid=(pl.cdiv(E, blk),),
        in_specs=[pl.BlockSpec(memory_space=pltpu.SMEM),
                  pl.BlockSpec((2, blk), lambda i: (0, i))],
        out_specs=[pl.BlockSpec((3, blk), lambda i: (0, i))] * nconv,
        out_shape=[jax.ShapeDtypeStruct((3, E), f32)] * nconv,
    )


@functools.cache
def _make_mm(Np, cin, cout, blk=256):
    kch = 3 * (cout // 2)

    def body(x_ref, w_ref, xw_ref, r_ref):
        y = jnp.dot(x_ref[...], w_ref[...], preferred_element_type=f32)
        xw_ref[0] = y[:, :kch]
        xw_ref[1] = y[:, kch:2 * kch]
        r_ref[...] = y[:, 2 * kch:]

    return pl.pallas_call(
        body,
        grid=(Np // blk,),
        in_specs=[pl.BlockSpec((blk, cin), lambda i: (i, 0)),
                  pl.BlockSpec((cin, 4 * cout), lambda i: (0, 0))],
        out_specs=[pl.BlockSpec((2, blk, kch), lambda i: (0, i, 0)),
                   pl.BlockSpec((blk, cout), lambda i: (i, 0))],
        out_shape=[jax.ShapeDtypeStruct((2, Np, kch), f32),
                   jax.ShapeDtypeStruct((Np, cout), f32)],
    )


@functools.cache
def _make_mmfin(Np, cp, cout, blk=256):
    hp = cp // 2
    kch = 3 * (cout // 2)

    def body(a0_ref, a1_ref, cnt_ref, r_ref, b_ref, w_ref, xw_ref, r2_ref):
        acc = jnp.concatenate([a0_ref[...], a1_ref[...]], axis=1)
        inv = 1.0 / jnp.maximum(cnt_ref[...], 1.0)
        hh = jnp.maximum(acc * inv + r_ref[...] + b_ref[...], 0.0)
        y = jnp.dot(hh, w_ref[...], preferred_element_type=f32)
        xw_ref[0] = y[:, :kch]
        xw_ref[1] = y[:, kch:2 * kch]
        r2_ref[...] = y[:, 2 * kch:]

    return pl.pallas_call(
        body,
        grid=(Np // blk,),
        in_specs=[pl.BlockSpec((blk, hp), lambda i: (i, 0)),
                  pl.BlockSpec((blk, hp), lambda i: (i, 0)),
                  pl.BlockSpec((blk, 1), lambda i: (i, 0)),
                  pl.BlockSpec((blk, cp), lambda i: (i, 0)),
                  pl.BlockSpec((1, cp), lambda i: (0, 0)),
                  pl.BlockSpec((cp, 4 * cout), lambda i: (0, 0))],
        out_specs=[pl.BlockSpec((2, blk, kch), lambda i: (0, i, 0)),
                   pl.BlockSpec((blk, cout), lambda i: (i, 0))],
        out_shape=[jax.ShapeDtypeStruct((2, Np, kch), f32),
                   jax.ShapeDtypeStruct((Np, cout), f32)],
    )


@functools.cache
def _make_fin(Np, cp, blk=256):
    hp = cp // 2

    def body(a0_ref, a1_ref, cnt_ref, r_ref, b_ref, h_ref):
        acc = jnp.concatenate([a0_ref[...], a1_ref[...]], axis=1)
        inv = 1.0 / jnp.maximum(cnt_ref[...], 1.0)
        h_ref[...] = jnp.maximum(acc * inv + r_ref[...] + b_ref[...], 0.0)

    return pl.pallas_call(
        body,
        grid=(Np // blk,),
        in_specs=[pl.BlockSpec((blk, hp), lambda i: (i, 0)),
                  pl.BlockSpec((blk, hp), lambda i: (i, 0)),
                  pl.BlockSpec((blk, 1), lambda i: (i, 0)),
                  pl.BlockSpec((blk, cp), lambda i: (i, 0)),
                  pl.BlockSpec((1, cp), lambda i: (0, 0))],
        out_specs=pl.BlockSpec((blk, cp), lambda i: (i, 0)),
        out_shape=jax.ShapeDtypeStruct((Np, cp), f32),
    )


@functools.cache
def _make_mm2(Np, c1, c2, cout, blk=256):
    kch = 3 * (cout // 2)

    def body(a_ref, b_ref, wa_ref, wb_ref, xw_ref, r_ref):
        y = (jnp.dot(a_ref[...], wa_ref[...], preferred_element_type=f32)
             + jnp.dot(b_ref[...], wb_ref[...], preferred_element_type=f32))
        xw_ref[0] = y[:, :kch]
        xw_ref[1] = y[:, kch:2 * kch]
        r_ref[...] = y[:, 2 * kch:]

    return pl.pallas_call(
        body,
        grid=(Np // blk,),
        in_specs=[pl.BlockSpec((blk, c1), lambda i: (i, 0)),
                  pl.BlockSpec((blk, c2), lambda i: (i, 0)),
                  pl.BlockSpec((c1, 4 * cout), lambda i: (0, 0)),
                  pl.BlockSpec((c2, 4 * cout), lambda i: (0, 0))],
        out_specs=[pl.BlockSpec((2, blk, kch), lambda i: (0, i, 0)),
                   pl.BlockSpec((blk, cout), lambda i: (i, 0))],
        out_shape=[jax.ShapeDtypeStruct((2, Np, kch), f32),
                   jax.ShapeDtypeStruct((Np, cout), f32)],
    )


@functools.cache
def _make_final(Np, cp=64, blk=256):
    hp = cp // 2

    def body(a0_ref, a1_ref, cnt_ref, r_ref, b_ref, fcw_ref, fcb_ref, o_ref):
        acc = jnp.concatenate([a0_ref[...], a1_ref[...]], axis=1)
        inv = 1.0 / jnp.maximum(cnt_ref[...], 1.0)
        hh = jnp.maximum(acc * inv + r_ref[...] + b_ref[...], 0.0)
        lg = jnp.dot(hh, fcw_ref[...], preferred_element_type=f32) + fcb_ref[...]
        mx = jnp.max(lg, axis=1, keepdims=True)
        lse = mx + jnp.log(jnp.sum(jnp.exp(lg - mx), axis=1, keepdims=True))
        o_ref[...] = lg - lse

    return pl.pallas_call(
        body,
        grid=(Np // blk,),
        in_specs=[pl.BlockSpec((blk, hp), lambda i: (i, 0)),
                  pl.BlockSpec((blk, hp), lambda i: (i, 0)),
                  pl.BlockSpec((blk, 1), lambda i: (i, 0)),
                  pl.BlockSpec((blk, cp), lambda i: (i, 0)),
                  pl.BlockSpec((1, cp), lambda i: (0, 0)),
                  pl.BlockSpec((cp, 2), lambda i: (0, 0)),
                  pl.BlockSpec((1, 2), lambda i: (0, 0))],
        out_specs=pl.BlockSpec((blk, 2), lambda i: (i, 0)),
        out_shape=jax.ShapeDtypeStruct((Np, 2), f32),
    )


def _gpar(ps):
    rows = []
    for p in ps:
        a = -0.5 / (p["sigma"] ** 2 + 1e-8)
        rows.append(jnp.concatenate([a, p["mu"]], axis=1))
    return jnp.concatenate(rows, axis=0)


def _wperm(W, root, cout):
    h = cout // 2
    order = np.array([k * cout + c * h + j
                      for c in range(2) for k in range(3) for j in range(h)])
    return jnp.concatenate([W[:, order], root], axis=1)


def kernel(x, edges_l6, ev6, edges_l5, ev5, pool_idx, unpool_idx, params):
    P = params
    src6 = edges_l6[0].astype(i32)
    dst6 = edges_l6[1].astype(i32)
    src5 = edges_l5[0].astype(i32)
    dst5 = edges_l5[1].astype(i32)

    cnt6r = _make_cnt(N6P, E6)(dst6)
    cnt6 = cnt6r[0, :, :1] + cnt6r[1, :, :1]
    cnt5r = _make_cnt(N5P, E5)(dst5)
    cnt5 = cnt5r[0, :, :1] + cnt5r[1, :, :1]

    g6 = _make_gk(E6, 4)(_gpar([P["c1"], P["c2"], P["c5"], P["c6"]]), ev6.T)
    g5 = _make_gk(E5, 2)(_gpar([P["c3"], P["c4"]]), ev5.T)
    g1, g2, gc5, gc6 = [g.reshape(-1) for g in g6]
    g3, g4 = [g.reshape(-1) for g in g5]

    agg6 = _make_agg(N6P, 64, E6, BE)
    agg5 = _make_agg(N5P, 128, E5, BE)

    xp = jnp.zeros((N6P, x.shape[1]), f32).at[:N6].set(x)
    b = {k: P[k]["b"].reshape(1, -1) for k in ("c1", "c2", "c3", "c4", "c5", "c6")}

    xw1, r1 = _make_mm(N6P, 32, 64)(xp, _wperm(P["c1"]["W"], P["c1"]["root"], 64))
    a1 = agg6(xw1.reshape(2 * N6P, -1), g1, src6, dst6)
    xw2, r2 = _make_mmfin(N6P, 64, 64)(a1[0], a1[1], cnt6, r1, b["c1"],
                                       _wperm(P["c2"]["W"], P["c2"]["root"], 64))
    a2 = agg6(xw2.reshape(2 * N6P, -1), g2, src6, dst6)
    skip = _make_fin(N6P, 64)(a2[0], a2[1], cnt6, r2, b["c2"])

    pidx = jnp.zeros((N5P * 7,), i32).at[:N5 * 7].set(
        pool_idx.astype(i32).reshape(-1))
    hp = _make_pool()(skip, pidx)

    xw3, r3 = _make_mm(N5P, 64, 128)(hp, _wperm(P["c3"]["W"], P["c3"]["root"], 128))
    a3 = agg5(xw3.reshape(2 * N5P, -1), g3, src5, dst5)
    xw4, r4 = _make_mmfin(N5P, 128, 128)(a3[0], a3[1], cnt5, r3, b["c3"],
                                         _wperm(P["c4"]["W"], P["c4"]["root"], 128))
    a4 = agg5(xw4.reshape(2 * N5P, -1), g4, src5, dst5)
    h4 = _make_fin(N5P, 128)(a4[0], a4[1], cnt5, r4, b["c4"])

    ar5 = jnp.arange(N5, dtype=i32)
    u0 = jnp.concatenate([ar5, unpool_idx[:, 0].astype(i32),
                          jnp.zeros((N6P - N6,), i32)])
    u1 = jnp.concatenate([ar5, unpool_idx[:, 1].astype(i32),
                          jnp.zeros((N6P - N6,), i32)])
    uf = jnp.stack([u0, u1], axis=1).reshape(-1)
    up = _make_unpool()(h4, uf)

    p5 = P["c5"]
    wa = _wperm(p5["W"][:128], p5["root"][:128], 64)
    wb = _wperm(p5["W"][128:], p5["root"][128:], 64)
    xw5, r5 = _make_mm2(N6P, 128, 64, 64)(up, skip, wa, wb)
    a5 = agg6(xw5.reshape(2 * N6P, -1), gc5, src6, dst6)
    xw6, r6 = _make_mmfin(N6P, 64, 64)(a5[0], a5[1], cnt6, r5, b["c5"],
                                       _wperm(P["c6"]["W"], P["c6"]["root"], 64))
    a6 = agg6(xw6.reshape(2 * N6P, -1), gc6, src6, dst6)

    out = _make_final(N6P)(a6[0], a6[1], cnt6, r6, b["c6"], P["fc_w"],
                           P["fc_b"].reshape(1, 2))
    return out[:N6]


# speedup vs baseline: 2.7706x; 1.0067x over previous
import functools

import numpy as np

import jax
import jax.numpy as jnp
from jax import lax
from jax.experimental import pallas as pl
from jax.experimental.pallas import tpu as pltpu
from jax.experimental.pallas import tpu_sc as plsc

N6, N5 = 40962, 10242
E6, E5 = 245760, 61440
N6P, N5P = 41472, 10752
BE = 64

f32 = jnp.float32
i32 = jnp.int32


def _sc_mesh():
    return plsc.VectorSubcoreMesh(core_axis_name="c", subcore_axis_name="s",
                                  num_cores=2, num_subcores=16)


@functools.cache
def _make_agg(Np, cout, E, be):
    h = cout // 2
    kch = 3 * h
    ept = E // 16
    nb = ept // be
    npt_sc = Np // 16
    zr = 48

    @functools.partial(
        pl.kernel, mesh=_sc_mesh(),
        out_type=jax.ShapeDtypeStruct((2, Np, h), f32),
        compiler_params=pltpu.CompilerParams(use_tc_tiling_on_sc=False),
        scratch_types=[
            pltpu.VMEM((be,), i32), pltpu.VMEM((be,), i32),
            pltpu.VMEM((be,), i32), pltpu.VMEM((be,), i32),
            pltpu.VMEM((be,), i32), pltpu.VMEM((be,), i32),
            pltpu.VMEM((3 * be,), f32), pltpu.VMEM((3 * be,), f32),
            pltpu.VMEM((3 * be,), f32), pltpu.VMEM((3 * be,), f32),
            pltpu.VMEM((be, kch), f32), pltpu.VMEM((be, kch), f32),
            pltpu.VMEM((be, h), f32), pltpu.VMEM((be, h), f32),
            pltpu.VMEM((zr, h), f32),
            pltpu.VMEM_SHARED((Np, h), f32),
            pltpu.SemaphoreType.DMA, pltpu.SemaphoreType.DMA,
            pltpu.SemaphoreType.DMA, pltpu.SemaphoreType.DMA,
            pltpu.SemaphoreType.DMA, pltpu.SemaphoreType.DMA,
        ],
    )
    def agg(xws, g3, srcs, dsts, out,
            ix0, ix1, dr0, dr1, dr2, dr3, gv0, gv1, gv2, gv3,
            rw0, rw1, mg0, mg1,
            zbuf, acc_sh, sm0, sm1, sg0, sg1, ss0, ss1):
        idxb, rwb, mgb = [ix0, ix1], [rw0, rw1], [mg0, mg1]
        gring = [gv0, gv1, gv2, gv3]
        dring = [dr0, dr1, dr2, dr3]
        smb, sgb, ssb = [sm0, sm1], [sg0, sg1], [ss0, ss1]
        c = lax.axis_index("c")
        s = lax.axis_index("s")
        ebase = s * ept
        n0sc = s * npt_sc
        cNp = c * Np

        def zstore(n, _):
            for cc in range(h // 16):
                zbuf[n, pl.ds(cc * 16, 16)] = jnp.zeros((16,), f32)
            return 0
        lax.fori_loop(0, zr, zstore, 0)

        def zcopy(zi, _):
            pltpu.sync_copy(zbuf, acc_sh.at[pl.ds(n0sc + zi * zr, zr)])
            return 0
        lax.fori_loop(0, npt_sc // zr, zcopy, 0)
        plsc.subcore_barrier()

        def issue_meta(bi, par, ring):
            eb = ebase + bi * be
            pltpu.async_copy(srcs.at[pl.ds(eb, be)], idxb[par], smb[par])
            pltpu.async_copy(dsts.at[pl.ds(eb, be)], dring[ring], smb[par])
            for kk in range(3):
                pltpu.async_copy(g3.at[pl.ds(kk * E + eb, be)],
                                 gring[ring].at[pl.ds(kk * be, be)], smb[par])

        def wait_meta(par, ring):
            pltpu.make_async_copy(srcs.at[pl.ds(0, be)], idxb[par],
                                  smb[par]).wait()
            pltpu.make_async_copy(dsts.at[pl.ds(0, be)], dring[ring],
                                  smb[par]).wait()
            for kk in range(3):
                pltpu.make_async_copy(g3.at[pl.ds(0, be)],
                                      gring[ring].at[pl.ds(kk * be, be)],
                                      smb[par]).wait()

        def compute(par, ring):
            def sub(sb, _):
                base = sb * 16
                g0 = gring[ring][pl.ds(base, 16)]
                g1 = gring[ring][pl.ds(be + base, 16)]
                g2 = gring[ring][pl.ds(2 * be + base, 16)]
                rows = rwb[par]
                msg = mgb[par]
                for j2 in range(16):
                    j = base + j2
                    a = g0[j2]
                    b2 = g1[j2]
                    cg = g2[j2]
                    for cc in range(h // 16):
                        o = cc * 16
                        msg[j, pl.ds(o, 16)] = (
                            rows[j, pl.ds(o, 16)] * a
                            + rows[j, pl.ds(h + o, 16)] * b2
                            + rows[j, pl.ds(2 * h + o, 16)] * cg)
                return 0
            lax.fori_loop(0, be // 16, sub, 0)

        issue_meta(0, 0, 0)

        def quad(qi, _):
            for q in range(4):
                b = qi * 4 + q
                par = q % 2
                opar = 1 - par

                @pl.when(b <= nb + 2)
                def _():
                    @pl.when(b >= 3)
                    def _():
                        pltpu.make_async_copy(
                            mgb[opar], acc_sh.at[dring[(q + 1) % 4]],
                            ssb[opar]).wait()

                    @pl.when(b < nb)
                    def _():
                        wait_meta(par, q)
                        for kk in range(be // 16):
                            idxb[par][pl.ds(kk * 16, 16)] = (
                                idxb[par][pl.ds(kk * 16, 16)] + cNp)
                        pltpu.async_copy(xws.at[idxb[par]], rwb[par], sgb[par])

                    @pl.when(jnp.logical_and(b >= 1, b <= nb))
                    def _():
                        pltpu.make_async_copy(xws.at[pl.ds(0, be)],
                                              rwb[opar], sgb[opar]).wait()

                    @pl.when(b + 1 < nb)
                    def _():
                        issue_meta(b + 1, opar, (q + 1) % 4)

                    @pl.when(jnp.logical_and(b >= 1, b <= nb))
                    def _():
                        compute(opar, (q + 3) % 4)
                        pltpu.async_copy(mgb[opar],
                                         acc_sh.at[dring[(q + 3) % 4]],
                                         ssb[opar], add=True)
            return 0
        lax.fori_loop(0, (nb + 3 + 3) // 4, quad, 0)

        plsc.subcore_barrier()
        pltpu.sync_copy(acc_sh.at[pl.ds(n0sc, npt_sc)],
                        out.at[c, pl.ds(n0sc, npt_sc)])

    return agg


@functools.cache
def _make_cnt(Np, E, be=128):
    ept = E // 32
    nb = ept // be
    npt_sc = Np // 16
    zr = 48

    @functools.partial(
        pl.kernel, mesh=_sc_mesh(),
        out_type=jax.ShapeDtypeStruct((2, Np, 16), f32),
        compiler_params=pltpu.CompilerParams(use_tc_tiling_on_sc=False),
        scratch_types=[
            pltpu.VMEM((be,), i32), pltpu.VMEM((be,), i32),
            pltpu.VMEM((be,), i32), pltpu.VMEM((be,), i32),
            pltpu.VMEM((be, 16), f32),
            pltpu.VMEM((zr, 16), f32),
            pltpu.VMEM_SHARED((Np, 16), f32),
            pltpu.SemaphoreType.DMA, pltpu.SemaphoreType.DMA,
            pltpu.SemaphoreType.DMA, pltpu.SemaphoreType.DMA,
        ],
    )
    def cntk(dsts, out, dr0, dr1, dr2, dr3, ones_v, zbuf, acc_sh,
             sm0, sm1, ss0, ss1):
        dring = [dr0, dr1, dr2, dr3]
        smb, ssb = [sm0, sm1], [ss0, ss1]
        c = lax.axis_index("c")
        s = lax.axis_index("s")
        wid = s * 2 + c
        ebase = wid * ept
        n0sc = s * npt_sc

        def fill(n, _):
            ones_v[n, pl.ds(0, 16)] = jnp.full((16,), 1.0, f32)
            for cc in range(1):
                pass
            zbuf[jnp.minimum(n, zr - 1), pl.ds(0, 16)] = jnp.zeros((16,), f32)
            return 0
        lax.fori_loop(0, be, fill, 0)

        def zcopy(zi, _):
            pltpu.sync_copy(zbuf, acc_sh.at[pl.ds(n0sc + zi * zr, zr)])
            return 0
        lax.fori_loop(0, npt_sc // zr, zcopy, 0)
        plsc.subcore_barrier()

        def issue_meta(bi, par, ring):
            pltpu.async_copy(dsts.at[pl.ds(ebase + bi * be, be)],
                             dring[ring], smb[par])

        issue_meta(0, 0, 0)

        def quad(qi, _):
            for q in range(4):
                b = qi * 4 + q
                par = q % 2

                @pl.when(b <= nb + 1)
                def _():
                    @pl.when(jnp.logical_and(b >= 2, b <= nb + 1))
                    def _():
                        pltpu.make_async_copy(ones_v,
                                              acc_sh.at[dring[(q + 2) % 4]],
                                              ssb[par]).wait()

                    @pl.when(b < nb)
                    def _():
                        pltpu.make_async_copy(dsts.at[pl.ds(0, be)],
                                              dring[q], smb[par]).wait()
                        pltpu.async_copy(ones_v, acc_sh.at[dring[q]],
                                         ssb[par], add=True)

                    @pl.when(b + 1 < nb)
                    def _():
                        issue_meta(b + 1, 1 - par, (q + 1) % 4)
            return 0
        lax.fori_loop(0, (nb + 2 + 3) // 4, quad, 0)

        plsc.subcore_barrier()
        pltpu.sync_copy(acc_sh.at[pl.ds(n0sc, npt_sc)],
                        out.at[c, pl.ds(n0sc, npt_sc)])

    return cntk


@functools.cache
def _make_pool():
    npt, C = N5P // 32, 64
    nbn = 16

    @functools.partial(
        pl.kernel, mesh=_sc_mesh(),
        out_type=jax.ShapeDtypeStruct((N5P, C), f32),
        compiler_params=pltpu.CompilerParams(use_tc_tiling_on_sc=False),
        scratch_types=[
            pltpu.VMEM((7 * nbn,), i32), pltpu.VMEM((7 * nbn,), i32),
            pltpu.VMEM((7 * nbn, C), f32), pltpu.VMEM((7 * nbn, C), f32),
            pltpu.VMEM((npt, C), f32),
            pltpu.SemaphoreType.DMA, pltpu.SemaphoreType.DMA,
            pltpu.SemaphoreType.DMA, pltpu.SemaphoreType.DMA,
        ],
    )
    def pool(skip, pidx, out, ix0, ix1, rw0, rw1, out_v, si0, si1, sg0, sg1):
        idxb, rwb = [ix0, ix1], [rw0, rw1]
        sib, sgb = [si0, si1], [sg0, sg1]
        wid = lax.axis_index("s") * 2 + lax.axis_index("c")
        n0 = wid * npt
        nb = npt // nbn

        def issue_idx(bi, par):
            pltpu.async_copy(pidx.at[pl.ds((n0 + bi * nbn) * 7, 7 * nbn)],
                             idxb[par], sib[par])

        issue_idx(0, 0)

        def pair(i, _):
            for par in range(2):
                b = i * 2 + par
                opar = 1 - par

                @pl.when(b <= nb)
                def _():
                    @pl.when(b < nb)
                    def _():
                        pltpu.make_async_copy(pidx.at[pl.ds(0, 7 * nbn)],
                                              idxb[par], sib[par]).wait()
                        pltpu.async_copy(skip.at[idxb[par]], rwb[par], sgb[par])

                    @pl.when(b >= 1)
                    def _():
                        pltpu.make_async_copy(skip.at[pl.ds(0, 7 * nbn)],
                                              rwb[opar], sgb[opar]).wait()

                    @pl.when(b + 1 < nb)
                    def _():
                        issue_idx(b + 1, opar)

                    @pl.when(b >= 1)
                    def _():
                        bm = b - 1
                        rows = rwb[opar]
                        for j in range(nbn):
                            for cc in range(C // 16):
                                o = cc * 16
                                v = rows[7 * j, pl.ds(o, 16)]
                                for t in range(1, 7):
                                    v = jnp.maximum(v, rows[7 * j + t,
                                                            pl.ds(o, 16)])
                                out_v[bm * nbn + j, pl.ds(o, 16)] = v
            return 0
        lax.fori_loop(0, (nb + 2) // 2, pair, 0)
        pltpu.sync_copy(out_v, out.at[pl.ds(n0, npt)])

    return pool


@functools.cache
def _make_unpool():
    npt, C = N6P // 32, 128
    nbr = 48

    @functools.partial(
        pl.kernel, mesh=_sc_mesh(),
        out_type=jax.ShapeDtypeStruct((N6P, C), f32),
        compiler_params=pltpu.CompilerParams(use_tc_tiling_on_sc=False),
        scratch_types=[
            pltpu.VMEM((2 * nbr,), i32), pltpu.VMEM((2 * nbr,), i32),
            pltpu.VMEM((2 * nbr, C), f32), pltpu.VMEM((2 * nbr, C), f32),
            pltpu.VMEM((nbr, C), f32), pltpu.VMEM((nbr, C), f32),
            pltpu.SemaphoreType.DMA, pltpu.SemaphoreType.DMA,
            pltpu.SemaphoreType.DMA, pltpu.SemaphoreType.DMA,
            pltpu.SemaphoreType.DMA, pltpu.SemaphoreType.DMA,
        ],
    )
    def unpool(h4, uf, out, ix0, ix1, rw0, rw1, ov0, ov1,
               si0, si1, sg0, sg1, so0, so1):
        idxb, rwb, ovb = [ix0, ix1], [rw0, rw1], [ov0, ov1]
        sib, sgb, sob = [si0, si1], [sg0, sg1], [so0, so1]
        wid = lax.axis_index("s") * 2 + lax.axis_index("c")
        n0 = wid * npt
        nb = npt // nbr

        def issue_idx(bi, par):
            pltpu.async_copy(uf.at[pl.ds((n0 + bi * nbr) * 2, 2 * nbr)],
                             idxb[par], sib[par])

        issue_idx(0, 0)

        def pair(i, _):
            for par in range(2):
                b = i * 2 + par
                opar = 1 - par

                @pl.when(b <= nb + 1)
                def _():
                    @pl.when(jnp.logical_and(b >= 2, b <= nb + 1))
                    def _():
                        pltpu.make_async_copy(ovb[par],
                                              out.at[pl.ds(0, nbr)],
                                              sob[par]).wait()

                    @pl.when(b < nb)
                    def _():
                        pltpu.make_async_copy(uf.at[pl.ds(0, 2 * nbr)],
                                              idxb[par], sib[par]).wait()
                        pltpu.async_copy(h4.at[idxb[par]], rwb[par], sgb[par])

                    @pl.when(jnp.logical_and(b >= 1, b <= nb))
                    def _():
                        pltpu.make_async_copy(h4.at[pl.ds(0, 2 * nbr)],
                                              rwb[opar], sgb[opar]).wait()

                    @pl.when(b + 1 < nb)
                    def _():
                        issue_idx(b + 1, opar)

                    @pl.when(jnp.logical_and(b >= 1, b <= nb))
                    def _():
                        bm = b - 1
                        rows = rwb[opar]

                        def row(j, _):
                            for cc in range(C // 16):
                                o = cc * 16
                                ovb[opar][j, pl.ds(o, 16)] = (
                                    rows[2 * j, pl.ds(o, 16)]
                                    + rows[2 * j + 1, pl.ds(o, 16)]) * 0.5
                            return 0
                        lax.fori_loop(0, nbr, row, 0)
                        pltpu.async_copy(ovb[opar],
                                         out.at[pl.ds(n0 + bm * nbr, nbr)],
                                         sob[opar])
            return 0
        lax.fori_loop(0, (nb + 3) // 2, pair, 0)

    return unpool


@functools.cache
def _make_gk(E, nconv, blk=2048):
    def body(par_ref, ev_ref, *out_refs):
        e0 = ev_ref[0:1, :]
        e1 = ev_ref[1:2, :]
        for ic in range(nconv):
            for k in range(3):
                a0 = par_ref[ic * 3 + k, 0]
                a1 = par_ref[ic * 3 + k, 1]
                d0 = e0 - par_ref[ic * 3 + k, 2]
                d1 = e1 - par_ref[ic * 3 + k, 3]
                out_refs[ic][k:k + 1, :] = jnp.exp(a0 * d0 * d0 + a1 * d1 * d1)

    return pl.pallas_call(
        body,
        grid=(pl.cdiv(E, blk),),
        in_specs=[pl.BlockSpec(memory_space=pltpu.SMEM),
                  pl.BlockSpec((2, blk), lambda i: (0, i))],
        out_specs=[pl.BlockSpec((3, blk), lambda i: (0, i))] * nconv,
        out_shape=[jax.ShapeDtypeStruct((3, E), f32)] * nconv,
    )


@functools.cache
def _make_mm(Np, cin, cout, blk=256):
    kch = 3 * (cout // 2)

    def body(x_ref, w_ref, xw_ref, r_ref):
        y = jnp.dot(x_ref[...], w_ref[...], preferred_element_type=f32)
        xw_ref[0] = y[:, :kch]
        xw_ref[1] = y[:, kch:2 * kch]
        r_ref[...] = y[:, 2 * kch:]

    return pl.pallas_call(
        body,
        grid=(Np // blk,),
        in_specs=[pl.BlockSpec((blk, cin), lambda i: (i, 0)),
                  pl.BlockSpec((cin, 4 * cout), lambda i: (0, 0))],
        out_specs=[pl.BlockSpec((2, blk, kch), lambda i: (0, i, 0)),
                   pl.BlockSpec((blk, cout), lambda i: (i, 0))],
        out_shape=[jax.ShapeDtypeStruct((2, Np, kch), f32),
                   jax.ShapeDtypeStruct((Np, cout), f32)],
    )


@functools.cache
def _make_mmfin(Np, cp, cout, blk=256):
    hp = cp // 2
    kch = 3 * (cout // 2)

    def body(a0_ref, a1_ref, cnt_ref, r_ref, b_ref, w_ref, xw_ref, r2_ref):
        acc = jnp.concatenate([a0_ref[...], a1_ref[...]], axis=1)
        inv = 1.0 / jnp.maximum(cnt_ref[...], 1.0)
        hh = jnp.maximum(acc * inv + r_ref[...] + b_ref[...], 0.0)
        y = jnp.dot(hh, w_ref[...], preferred_element_type=f32)
        xw_ref[0] = y[:, :kch]
        xw_ref[1] = y[:, kch:2 * kch]
        r2_ref[...] = y[:, 2 * kch:]

    return pl.pallas_call(
        body,
        grid=(Np // blk,),
        in_specs=[pl.BlockSpec((blk, hp), lambda i: (i, 0)),
                  pl.BlockSpec((blk, hp), lambda i: (i, 0)),
                  pl.BlockSpec((blk, 1), lambda i: (i, 0)),
                  pl.BlockSpec((blk, cp), lambda i: (i, 0)),
                  pl.BlockSpec((1, cp), lambda i: (0, 0)),
                  pl.BlockSpec((cp, 4 * cout), lambda i: (0, 0))],
        out_specs=[pl.BlockSpec((2, blk, kch), lambda i: (0, i, 0)),
                   pl.BlockSpec((blk, cout), lambda i: (i, 0))],
        out_shape=[jax.ShapeDtypeStruct((2, Np, kch), f32),
                   jax.ShapeDtypeStruct((Np, cout), f32)],
    )


@functools.cache
def _make_fin(Np, cp, blk=256):
    hp = cp // 2

    def body(a0_ref, a1_ref, cnt_ref, r_ref, b_ref, h_ref):
        acc = jnp.concatenate([a0_ref[...], a1_ref[...]], axis=1)
        inv = 1.0 / jnp.maximum(cnt_ref[...], 1.0)
        h_ref[...] = jnp.maximum(acc * inv + r_ref[...] + b_ref[...], 0.0)

    return pl.pallas_call(
        body,
        grid=(Np // blk,),
        in_specs=[pl.BlockSpec((blk, hp), lambda i: (i, 0)),
                  pl.BlockSpec((blk, hp), lambda i: (i, 0)),
                  pl.BlockSpec((blk, 1), lambda i: (i, 0)),
                  pl.BlockSpec((blk, cp), lambda i: (i, 0)),
                  pl.BlockSpec((1, cp), lambda i: (0, 0))],
        out_specs=pl.BlockSpec((blk, cp), lambda i: (i, 0)),
        out_shape=jax.ShapeDtypeStruct((Np, cp), f32),
    )


@functools.cache
def _make_mm2(Np, c1, c2, cout, blk=256):
    kch = 3 * (cout // 2)

    def body(a_ref, b_ref, wa_ref, wb_ref, xw_ref, r_ref):
        y = (jnp.dot(a_ref[...], wa_ref[...], preferred_element_type=f32)
             + jnp.dot(b_ref[...], wb_ref[...], preferred_element_type=f32))
        xw_ref[0] = y[:, :kch]
        xw_ref[1] = y[:, kch:2 * kch]
        r_ref[...] = y[:, 2 * kch:]

    return pl.pallas_call(
        body,
        grid=(Np // blk,),
        in_specs=[pl.BlockSpec((blk, c1), lambda i: (i, 0)),
                  pl.BlockSpec((blk, c2), lambda i: (i, 0)),
                  pl.BlockSpec((c1, 4 * cout), lambda i: (0, 0)),
                  pl.BlockSpec((c2, 4 * cout), lambda i: (0, 0))],
        out_specs=[pl.BlockSpec((2, blk, kch), lambda i: (0, i, 0)),
                   pl.BlockSpec((blk, cout), lambda i: (i, 0))],
        out_shape=[jax.ShapeDtypeStruct((2, Np, kch), f32),
                   jax.ShapeDtypeStruct((Np, cout), f32)],
    )


@functools.cache
def _make_final(Np, cp=64, blk=256):
    hp = cp // 2

    def body(a0_ref, a1_ref, cnt_ref, r_ref, b_ref, fcw_ref, fcb_ref, o_ref):
        acc = jnp.concatenate([a0_ref[...], a1_ref[...]], axis=1)
        inv = 1.0 / jnp.maximum(cnt_ref[...], 1.0)
        hh = jnp.maximum(acc * inv + r_ref[...] + b_ref[...], 0.0)
        lg = jnp.dot(hh, fcw_ref[...], preferred_element_type=f32) + fcb_ref[...]
        mx = jnp.max(lg, axis=1, keepdims=True)
        lse = mx + jnp.log(jnp.sum(jnp.exp(lg - mx), axis=1, keepdims=True))
        o_ref[...] = lg - lse

    return pl.pallas_call(
        body,
        grid=(Np // blk,),
        in_specs=[pl.BlockSpec((blk, hp), lambda i: (i, 0)),
                  pl.BlockSpec((blk, hp), lambda i: (i, 0)),
                  pl.BlockSpec((blk, 1), lambda i: (i, 0)),
                  pl.BlockSpec((blk, cp), lambda i: (i, 0)),
                  pl.BlockSpec((1, cp), lambda i: (0, 0)),
                  pl.BlockSpec((cp, 2), lambda i: (0, 0)),
                  pl.BlockSpec((1, 2), lambda i: (0, 0))],
        out_specs=pl.BlockSpec((blk, 2), lambda i: (i, 0)),
        out_shape=jax.ShapeDtypeStruct((Np, 2), f32),
    )


def _gpar(ps):
    rows = []
    for p in ps:
        a = -0.5 / (p["sigma"] ** 2 + 1e-8)
        rows.append(jnp.concatenate([a, p["mu"]], axis=1))
    return jnp.concatenate(rows, axis=0)


def _wperm(W, root, cout):
    h = cout // 2
    order = np.array([k * cout + c * h + j
                      for c in range(2) for k in range(3) for j in range(h)])
    return jnp.concatenate([W[:, order], root], axis=1)


def kernel(x, edges_l6, ev6, edges_l5, ev5, pool_idx, unpool_idx, params):
    P = params
    src6 = edges_l6[0].astype(i32)
    dst6 = edges_l6[1].astype(i32)
    src5 = edges_l5[0].astype(i32)
    dst5 = edges_l5[1].astype(i32)

    cnt6r = _make_cnt(N6P, E6)(dst6)
    cnt6 = cnt6r[0, :, :1] + cnt6r[1, :, :1]
    cnt5r = _make_cnt(N5P, E5)(dst5)
    cnt5 = cnt5r[0, :, :1] + cnt5r[1, :, :1]

    g6 = _make_gk(E6, 4)(_gpar([P["c1"], P["c2"], P["c5"], P["c6"]]), ev6.T)
    g5 = _make_gk(E5, 2)(_gpar([P["c3"], P["c4"]]), ev5.T)
    g1, g2, gc5, gc6 = [g.reshape(-1) for g in g6]
    g3, g4 = [g.reshape(-1) for g in g5]

    agg6 = _make_agg(N6P, 64, E6, 128)
    agg5 = _make_agg(N5P, 128, E5, BE)

    xp = jnp.zeros((N6P, x.shape[1]), f32).at[:N6].set(x)
    b = {k: P[k]["b"].reshape(1, -1) for k in ("c1", "c2", "c3", "c4", "c5", "c6")}

    xw1, r1 = _make_mm(N6P, 32, 64)(xp, _wperm(P["c1"]["W"], P["c1"]["root"], 64))
    a1 = agg6(xw1.reshape(2 * N6P, -1), g1, src6, dst6)
    xw2, r2 = _make_mmfin(N6P, 64, 64)(a1[0], a1[1], cnt6, r1, b["c1"],
                                       _wperm(P["c2"]["W"], P["c2"]["root"], 64))
    a2 = agg6(xw2.reshape(2 * N6P, -1), g2, src6, dst6)
    skip = _make_fin(N6P, 64)(a2[0], a2[1], cnt6, r2, b["c2"])

    pidx = jnp.zeros((N5P * 7,), i32).at[:N5 * 7].set(
        pool_idx.astype(i32).reshape(-1))
    hp = _make_pool()(skip, pidx)

    xw3, r3 = _make_mm(N5P, 64, 128)(hp, _wperm(P["c3"]["W"], P["c3"]["root"], 128))
    a3 = agg5(xw3.reshape(2 * N5P, -1), g3, src5, dst5)
    xw4, r4 = _make_mmfin(N5P, 128, 128)(a3[0], a3[1], cnt5, r3, b["c3"],
                                         _wperm(P["c4"]["W"], P["c4"]["root"], 128))
    a4 = agg5(xw4.reshape(2 * N5P, -1), g4, src5, dst5)
    h4 = _make_fin(N5P, 128)(a4[0], a4[1], cnt5, r4, b["c4"])

    ar5 = jnp.arange(N5, dtype=i32)
    u0 = jnp.concatenate([ar5, unpool_idx[:, 0].astype(i32),
                          jnp.zeros((N6P - N6,), i32)])
    u1 = jnp.concatenate([ar5, unpool_idx[:, 1].astype(i32),
                          jnp.zeros((N6P - N6,), i32)])
    uf = jnp.stack([u0, u1], axis=1).reshape(-1)
    up = _make_unpool()(h4, uf)

    p5 = P["c5"]
    wa = _wperm(p5["W"][:128], p5["root"][:128], 64)
    wb = _wperm(p5["W"][128:], p5["root"][128:], 64)
    xw5, r5 = _make_mm2(N6P, 128, 64, 64)(up, skip, wa, wb)
    a5 = agg6(xw5.reshape(2 * N6P, -1), gc5, src6, dst6)
    xw6, r6 = _make_mmfin(N6P, 64, 64)(a5[0], a5[1], cnt6, r5, b["c5"],
                                       _wperm(P["c6"]["W"], P["c6"]["root"], 64))
    a6 = agg6(xw6.reshape(2 * N6P, -1), gc6, src6, dst6)

    out = _make_final(N6P)(a6[0], a6[1], cnt6, r6, b["c6"], P["fc_w"],
                           P["fc_b"].reshape(1, 2))
    return out[:N6]


# speedup vs baseline: 3.0794x; 1.1114x over previous
import functools

import numpy as np

import jax
import jax.numpy as jnp
from jax import lax
from jax.experimental import pallas as pl
from jax.experimental.pallas import tpu as pltpu
from jax.experimental.pallas import tpu_sc as plsc

N6, N5 = 40962, 10242
E6, E5 = 245760, 61440
N6P, N5P = 41472, 10752
BE = 64

f32 = jnp.float32
i32 = jnp.int32


def _sc_mesh():
    return plsc.VectorSubcoreMesh(core_axis_name="c", subcore_axis_name="s",
                                  num_cores=2, num_subcores=16)


@functools.cache
def _make_agg(Np, cout, E, be):
    h = cout // 2
    kch = 3 * h
    ept = E // 16
    nb = ept // be
    npt_sc = Np // 16
    zr = 48

    @functools.partial(
        pl.kernel, mesh=_sc_mesh(),
        out_type=jax.ShapeDtypeStruct((2, Np, h), f32),
        compiler_params=pltpu.CompilerParams(use_tc_tiling_on_sc=False),
        scratch_types=[
            pltpu.VMEM((be,), i32), pltpu.VMEM((be,), i32),
            pltpu.VMEM((be,), i32), pltpu.VMEM((be,), i32),
            pltpu.VMEM((be,), i32), pltpu.VMEM((be,), i32),
            pltpu.VMEM((3 * be,), f32), pltpu.VMEM((3 * be,), f32),
            pltpu.VMEM((3 * be,), f32), pltpu.VMEM((3 * be,), f32),
            pltpu.VMEM((be, kch), f32), pltpu.VMEM((be, kch), f32),
            pltpu.VMEM((be, h), f32), pltpu.VMEM((be, h), f32),
            pltpu.VMEM((zr, h), f32),
            pltpu.VMEM_SHARED((Np, h), f32),
            pltpu.SemaphoreType.DMA, pltpu.SemaphoreType.DMA,
            pltpu.SemaphoreType.DMA, pltpu.SemaphoreType.DMA,
            pltpu.SemaphoreType.DMA, pltpu.SemaphoreType.DMA,
        ],
    )
    def agg(xws, g3, srcs, dsts, out,
            ix0, ix1, dr0, dr1, dr2, dr3, gv0, gv1, gv2, gv3,
            rw0, rw1, mg0, mg1,
            zbuf, acc_sh, sm0, sm1, sg0, sg1, ss0, ss1):
        idxb, rwb, mgb = [ix0, ix1], [rw0, rw1], [mg0, mg1]
        gring = [gv0, gv1, gv2, gv3]
        dring = [dr0, dr1, dr2, dr3]
        smb, sgb, ssb = [sm0, sm1], [sg0, sg1], [ss0, ss1]
        c = lax.axis_index("c")
        s = lax.axis_index("s")
        ebase = s * ept
        n0sc = s * npt_sc
        cNp = c * Np

        def zstore(n, _):
            for cc in range(h // 16):
                zbuf[n, pl.ds(cc * 16, 16)] = jnp.zeros((16,), f32)
            return 0
        lax.fori_loop(0, zr, zstore, 0)

        def zcopy(zi, _):
            pltpu.sync_copy(zbuf, acc_sh.at[pl.ds(n0sc + zi * zr, zr)])
            return 0
        lax.fori_loop(0, npt_sc // zr, zcopy, 0)
        plsc.subcore_barrier()

        def issue_meta(bi, par, ring):
            eb = ebase + bi * be
            pltpu.async_copy(srcs.at[pl.ds(eb, be)], idxb[par], smb[par])
            pltpu.async_copy(dsts.at[pl.ds(eb, be)], dring[ring], smb[par])
            for kk in range(3):
                pltpu.async_copy(g3.at[pl.ds(kk * E + eb, be)],
                                 gring[ring].at[pl.ds(kk * be, be)], smb[par])

        def wait_meta(par, ring):
            pltpu.make_async_copy(srcs.at[pl.ds(0, be)], idxb[par],
                                  smb[par]).wait()
            pltpu.make_async_copy(dsts.at[pl.ds(0, be)], dring[ring],
                                  smb[par]).wait()
            for kk in range(3):
                pltpu.make_async_copy(g3.at[pl.ds(0, be)],
                                      gring[ring].at[pl.ds(kk * be, be)],
                                      smb[par]).wait()

        def compute(par, ring):
            def sub(sb, _):
                base = sb * 16
                g0 = gring[ring][pl.ds(base, 16)]
                g1 = gring[ring][pl.ds(be + base, 16)]
                g2 = gring[ring][pl.ds(2 * be + base, 16)]
                rows = rwb[par]
                msg = mgb[par]
                for j2 in range(16):
                    j = base + j2
                    a = g0[j2]
                    b2 = g1[j2]
                    cg = g2[j2]
                    for cc in range(h // 16):
                        o = cc * 16
                        msg[j, pl.ds(o, 16)] = (
                            rows[j, pl.ds(o, 16)] * a
                            + rows[j, pl.ds(h + o, 16)] * b2
                            + rows[j, pl.ds(2 * h + o, 16)] * cg)
                return 0
            lax.fori_loop(0, be // 16, sub, 0)

        issue_meta(0, 0, 0)

        def quad(qi, _):
            for q in range(4):
                b = qi * 4 + q
                par = q % 2
                opar = 1 - par

                @pl.when(b <= nb + 2)
                def _():
                    @pl.when(b >= 3)
                    def _():
                        pltpu.make_async_copy(
                            mgb[opar], acc_sh.at[dring[(q + 1) % 4]],
                            ssb[opar]).wait()

                    @pl.when(b < nb)
                    def _():
                        wait_meta(par, q)
                        for kk in range(be // 16):
                            idxb[par][pl.ds(kk * 16, 16)] = (
                                idxb[par][pl.ds(kk * 16, 16)] + cNp)
                        pltpu.async_copy(xws.at[idxb[par]], rwb[par], sgb[par])

                    @pl.when(jnp.logical_and(b >= 1, b <= nb))
                    def _():
                        pltpu.make_async_copy(xws.at[pl.ds(0, be)],
                                              rwb[opar], sgb[opar]).wait()

                    @pl.when(b + 1 < nb)
                    def _():
                        issue_meta(b + 1, opar, (q + 1) % 4)

                    @pl.when(jnp.logical_and(b >= 1, b <= nb))
                    def _():
                        compute(opar, (q + 3) % 4)
                        pltpu.async_copy(mgb[opar],
                                         acc_sh.at[dring[(q + 3) % 4]],
                                         ssb[opar], add=True)
            return 0
        lax.fori_loop(0, (nb + 3 + 3) // 4, quad, 0)

        plsc.subcore_barrier()
        pltpu.sync_copy(acc_sh.at[pl.ds(n0sc, npt_sc)],
                        out.at[c, pl.ds(n0sc, npt_sc)])

    return agg


@functools.cache
def _make_cnt(Np, E, be=128):
    ept = E // 32
    nb = ept // be
    npt_sc = Np // 16
    zr = 48

    @functools.partial(
        pl.kernel, mesh=_sc_mesh(),
        out_type=jax.ShapeDtypeStruct((2, Np, 16), f32),
        compiler_params=pltpu.CompilerParams(use_tc_tiling_on_sc=False),
        scratch_types=[
            pltpu.VMEM((be,), i32), pltpu.VMEM((be,), i32),
            pltpu.VMEM((be,), i32), pltpu.VMEM((be,), i32),
            pltpu.VMEM((be, 16), f32),
            pltpu.VMEM((zr, 16), f32),
            pltpu.VMEM_SHARED((Np, 16), f32),
            pltpu.SemaphoreType.DMA, pltpu.SemaphoreType.DMA,
            pltpu.SemaphoreType.DMA, pltpu.SemaphoreType.DMA,
        ],
    )
    def cntk(dsts, out, dr0, dr1, dr2, dr3, ones_v, zbuf, acc_sh,
             sm0, sm1, ss0, ss1):
        dring = [dr0, dr1, dr2, dr3]
        smb, ssb = [sm0, sm1], [ss0, ss1]
        c = lax.axis_index("c")
        s = lax.axis_index("s")
        wid = s * 2 + c
        ebase = wid * ept
        n0sc = s * npt_sc

        def fill(n, _):
            ones_v[n, pl.ds(0, 16)] = jnp.full((16,), 1.0, f32)
            for cc in range(1):
                pass
            zbuf[jnp.minimum(n, zr - 1), pl.ds(0, 16)] = jnp.zeros((16,), f32)
            return 0
        lax.fori_loop(0, be, fill, 0)

        def zcopy(zi, _):
            pltpu.sync_copy(zbuf, acc_sh.at[pl.ds(n0sc + zi * zr, zr)])
            return 0
        lax.fori_loop(0, npt_sc // zr, zcopy, 0)
        plsc.subcore_barrier()

        def issue_meta(bi, par, ring):
            pltpu.async_copy(dsts.at[pl.ds(ebase + bi * be, be)],
                             dring[ring], smb[par])

        issue_meta(0, 0, 0)

        def quad(qi, _):
            for q in range(4):
                b = qi * 4 + q
                par = q % 2

                @pl.when(b <= nb + 1)
                def _():
                    @pl.when(jnp.logical_and(b >= 2, b <= nb + 1))
                    def _():
                        pltpu.make_async_copy(ones_v,
                                              acc_sh.at[dring[(q + 2) % 4]],
                                              ssb[par]).wait()

                    @pl.when(b < nb)
                    def _():
                        pltpu.make_async_copy(dsts.at[pl.ds(0, be)],
                                              dring[q], smb[par]).wait()
                        pltpu.async_copy(ones_v, acc_sh.at[dring[q]],
                                         ssb[par], add=True)

                    @pl.when(b + 1 < nb)
                    def _():
                        issue_meta(b + 1, 1 - par, (q + 1) % 4)
            return 0
        lax.fori_loop(0, (nb + 2 + 3) // 4, quad, 0)

        plsc.subcore_barrier()
        pltpu.sync_copy(acc_sh.at[pl.ds(n0sc, npt_sc)],
                        out.at[c, pl.ds(n0sc, npt_sc)])

    return cntk


@functools.cache
def _make_pool():
    npt, C = N5P // 32, 64
    nbn = 16

    @functools.partial(
        pl.kernel, mesh=_sc_mesh(),
        out_type=jax.ShapeDtypeStruct((N5P, C), f32),
        compiler_params=pltpu.CompilerParams(use_tc_tiling_on_sc=False),
        scratch_types=[
            pltpu.VMEM((7 * nbn,), i32), pltpu.VMEM((7 * nbn,), i32),
            pltpu.VMEM((7 * nbn, C), f32), pltpu.VMEM((7 * nbn, C), f32),
            pltpu.VMEM((npt, C), f32),
            pltpu.SemaphoreType.DMA, pltpu.SemaphoreType.DMA,
            pltpu.SemaphoreType.DMA, pltpu.SemaphoreType.DMA,
        ],
    )
    def pool(skip, pidx, out, ix0, ix1, rw0, rw1, out_v, si0, si1, sg0, sg1):
        idxb, rwb = [ix0, ix1], [rw0, rw1]
        sib, sgb = [si0, si1], [sg0, sg1]
        wid = lax.axis_index("s") * 2 + lax.axis_index("c")
        n0 = wid * npt
        nb = npt // nbn

        def issue_idx(bi, par):
            pltpu.async_copy(pidx.at[pl.ds((n0 + bi * nbn) * 7, 7 * nbn)],
                             idxb[par], sib[par])

        issue_idx(0, 0)

        def pair(i, _):
            for par in range(2):
                b = i * 2 + par
                opar = 1 - par

                @pl.when(b <= nb)
                def _():
                    @pl.when(b < nb)
                    def _():
                        pltpu.make_async_copy(pidx.at[pl.ds(0, 7 * nbn)],
                                              idxb[par], sib[par]).wait()
                        pltpu.async_copy(skip.at[idxb[par]], rwb[par], sgb[par])

                    @pl.when(b >= 1)
                    def _():
                        pltpu.make_async_copy(skip.at[pl.ds(0, 7 * nbn)],
                                              rwb[opar], sgb[opar]).wait()

                    @pl.when(b + 1 < nb)
                    def _():
                        issue_idx(b + 1, opar)

                    @pl.when(b >= 1)
                    def _():
                        bm = b - 1
                        rows = rwb[opar]
                        for j in range(nbn):
                            for cc in range(C // 16):
                                o = cc * 16
                                v = rows[7 * j, pl.ds(o, 16)]
                                for t in range(1, 7):
                                    v = jnp.maximum(v, rows[7 * j + t,
                                                            pl.ds(o, 16)])
                                out_v[bm * nbn + j, pl.ds(o, 16)] = v
            return 0
        lax.fori_loop(0, (nb + 2) // 2, pair, 0)
        pltpu.sync_copy(out_v, out.at[pl.ds(n0, npt)])

    return pool


@functools.cache
def _make_unpool():
    npt, C = N6P // 32, 128
    nbr = 48

    @functools.partial(
        pl.kernel, mesh=_sc_mesh(),
        out_type=jax.ShapeDtypeStruct((N6P, C), f32),
        compiler_params=pltpu.CompilerParams(use_tc_tiling_on_sc=False),
        scratch_types=[
            pltpu.VMEM((2 * nbr,), i32), pltpu.VMEM((2 * nbr,), i32),
            pltpu.VMEM((2 * nbr, C), f32), pltpu.VMEM((2 * nbr, C), f32),
            pltpu.VMEM((nbr, C), f32), pltpu.VMEM((nbr, C), f32),
            pltpu.SemaphoreType.DMA, pltpu.SemaphoreType.DMA,
            pltpu.SemaphoreType.DMA, pltpu.SemaphoreType.DMA,
            pltpu.SemaphoreType.DMA, pltpu.SemaphoreType.DMA,
        ],
    )
    def unpool(h4, uf, out, ix0, ix1, rw0, rw1, ov0, ov1,
               si0, si1, sg0, sg1, so0, so1):
        idxb, rwb, ovb = [ix0, ix1], [rw0, rw1], [ov0, ov1]
        sib, sgb, sob = [si0, si1], [sg0, sg1], [so0, so1]
        wid = lax.axis_index("s") * 2 + lax.axis_index("c")
        n0 = wid * npt
        nb = npt // nbr

        def issue_idx(bi, par):
            pltpu.async_copy(uf.at[pl.ds((n0 + bi * nbr) * 2, 2 * nbr)],
                             idxb[par], sib[par])

        issue_idx(0, 0)

        def pair(i, _):
            for par in range(2):
                b = i * 2 + par
                opar = 1 - par

                @pl.when(b <= nb + 1)
                def _():
                    @pl.when(jnp.logical_and(b >= 2, b <= nb + 1))
                    def _():
                        pltpu.make_async_copy(ovb[par],
                                              out.at[pl.ds(0, nbr)],
                                              sob[par]).wait()

                    @pl.when(b < nb)
                    def _():
                        pltpu.make_async_copy(uf.at[pl.ds(0, 2 * nbr)],
                                              idxb[par], sib[par]).wait()
                        pltpu.async_copy(h4.at[idxb[par]], rwb[par], sgb[par])

                    @pl.when(jnp.logical_and(b >= 1, b <= nb))
                    def _():
                        pltpu.make_async_copy(h4.at[pl.ds(0, 2 * nbr)],
                                              rwb[opar], sgb[opar]).wait()

                    @pl.when(b + 1 < nb)
                    def _():
                        issue_idx(b + 1, opar)

                    @pl.when(jnp.logical_and(b >= 1, b <= nb))
                    def _():
                        bm = b - 1
                        rows = rwb[opar]

                        def row(j, _):
                            for cc in range(C // 16):
                                o = cc * 16
                                ovb[opar][j, pl.ds(o, 16)] = (
                                    rows[2 * j, pl.ds(o, 16)]
                                    + rows[2 * j + 1, pl.ds(o, 16)]) * 0.5
                            return 0
                        lax.fori_loop(0, nbr, row, 0)
                        pltpu.async_copy(ovb[opar],
                                         out.at[pl.ds(n0 + bm * nbr, nbr)],
                                         sob[opar])
            return 0
        lax.fori_loop(0, (nb + 3) // 2, pair, 0)

    return unpool


@functools.cache
def _make_gk(E, nconv, blk=2048):
    def body(par_ref, ev_ref, *out_refs):
        e0 = ev_ref[0:1, :]
        e1 = ev_ref[1:2, :]
        for ic in range(nconv):
            for k in range(3):
                a0 = par_ref[ic * 3 + k, 0]
                a1 = par_ref[ic * 3 + k, 1]
                d0 = e0 - par_ref[ic * 3 + k, 2]
                d1 = e1 - par_ref[ic * 3 + k, 3]
                out_refs[ic][k:k + 1, :] = jnp.exp(a0 * d0 * d0 + a1 * d1 * d1)

    return pl.pallas_call(
        body,
        grid=(pl.cdiv(E, blk),),
        in_specs=[pl.BlockSpec(memory_space=pltpu.SMEM),
                  pl.BlockSpec((2, blk), lambda i: (0, i))],
        out_specs=[pl.BlockSpec((3, blk), lambda i: (0, i))] * nconv,
        out_shape=[jax.ShapeDtypeStruct((3, E), f32)] * nconv,
    )


@functools.cache
def _make_mm(Np, cin, cout, blk=512):
    kch = 3 * (cout // 2)

    def body(x_ref, w_ref, xw_ref, r_ref):
        y = jnp.dot(x_ref[...], w_ref[...], preferred_element_type=f32)
        xw_ref[0] = y[:, :kch]
        xw_ref[1] = y[:, kch:2 * kch]
        r_ref[...] = y[:, 2 * kch:]

    return pl.pallas_call(
        body,
        grid=(Np // blk,),
        in_specs=[pl.BlockSpec((blk, cin), lambda i: (i, 0)),
                  pl.BlockSpec((cin, 4 * cout), lambda i: (0, 0))],
        out_specs=[pl.BlockSpec((2, blk, kch), lambda i: (0, i, 0)),
                   pl.BlockSpec((blk, cout), lambda i: (i, 0))],
        out_shape=[jax.ShapeDtypeStruct((2, Np, kch), f32),
                   jax.ShapeDtypeStruct((Np, cout), f32)],
    )


@functools.cache
def _make_mmfin(Np, cp, cout, blk=512):
    hp = cp // 2
    kch = 3 * (cout // 2)

    def body(a0_ref, a1_ref, cnt_ref, r_ref, b_ref, w_ref, xw_ref, r2_ref):
        acc = jnp.concatenate([a0_ref[...], a1_ref[...]], axis=1)
        inv = 1.0 / jnp.maximum(cnt_ref[...], 1.0)
        hh = jnp.maximum(acc * inv + r_ref[...] + b_ref[...], 0.0)
        y = jnp.dot(hh, w_ref[...], preferred_element_type=f32)
        xw_ref[0] = y[:, :kch]
        xw_ref[1] = y[:, kch:2 * kch]
        r2_ref[...] = y[:, 2 * kch:]

    return pl.pallas_call(
        body,
        grid=(Np // blk,),
        in_specs=[pl.BlockSpec((blk, hp), lambda i: (i, 0)),
                  pl.BlockSpec((blk, hp), lambda i: (i, 0)),
                  pl.BlockSpec((blk, 1), lambda i: (i, 0)),
                  pl.BlockSpec((blk, cp), lambda i: (i, 0)),
                  pl.BlockSpec((1, cp), lambda i: (0, 0)),
                  pl.BlockSpec((cp, 4 * cout), lambda i: (0, 0))],
        out_specs=[pl.BlockSpec((2, blk, kch), lambda i: (0, i, 0)),
                   pl.BlockSpec((blk, cout), lambda i: (i, 0))],
        out_shape=[jax.ShapeDtypeStruct((2, Np, kch), f32),
                   jax.ShapeDtypeStruct((Np, cout), f32)],
    )


@functools.cache
def _make_fin(Np, cp, blk=512):
    hp = cp // 2

    def body(a0_ref, a1_ref, cnt_ref, r_ref, b_ref, h_ref):
        acc = jnp.concatenate([a0_ref[...], a1_ref[...]], axis=1)
        inv = 1.0 / jnp.maximum(cnt_ref[...], 1.0)
        h_ref[...] = jnp.maximum(acc * inv + r_ref[...] + b_ref[...], 0.0)

    return pl.pallas_call(
        body,
        grid=(Np // blk,),
        in_specs=[pl.BlockSpec((blk, hp), lambda i: (i, 0)),
                  pl.BlockSpec((blk, hp), lambda i: (i, 0)),
                  pl.BlockSpec((blk, 1), lambda i: (i, 0)),
                  pl.BlockSpec((blk, cp), lambda i: (i, 0)),
                  pl.BlockSpec((1, cp), lambda i: (0, 0))],
        out_specs=pl.BlockSpec((blk, cp), lambda i: (i, 0)),
        out_shape=jax.ShapeDtypeStruct((Np, cp), f32),
    )


@functools.cache
def _make_mm2(Np, c1, c2, cout, blk=512):
    kch = 3 * (cout // 2)

    def body(a_ref, b_ref, wa_ref, wb_ref, xw_ref, r_ref):
        y = (jnp.dot(a_ref[...], wa_ref[...], preferred_element_type=f32)
             + jnp.dot(b_ref[...], wb_ref[...], preferred_element_type=f32))
        xw_ref[0] = y[:, :kch]
        xw_ref[1] = y[:, kch:2 * kch]
        r_ref[...] = y[:, 2 * kch:]

    return pl.pallas_call(
        body,
        grid=(Np // blk,),
        in_specs=[pl.BlockSpec((blk, c1), lambda i: (i, 0)),
                  pl.BlockSpec((blk, c2), lambda i: (i, 0)),
                  pl.BlockSpec((c1, 4 * cout), lambda i: (0, 0)),
                  pl.BlockSpec((c2, 4 * cout), lambda i: (0, 0))],
        out_specs=[pl.BlockSpec((2, blk, kch), lambda i: (0, i, 0)),
                   pl.BlockSpec((blk, cout), lambda i: (i, 0))],
        out_shape=[jax.ShapeDtypeStruct((2, Np, kch), f32),
                   jax.ShapeDtypeStruct((Np, cout), f32)],
    )


@functools.cache
def _make_final(Np, cp=64, blk=512):
    hp = cp // 2

    def body(a0_ref, a1_ref, cnt_ref, r_ref, b_ref, fcw_ref, fcb_ref, o_ref):
        acc = jnp.concatenate([a0_ref[...], a1_ref[...]], axis=1)
        inv = 1.0 / jnp.maximum(cnt_ref[...], 1.0)
        hh = jnp.maximum(acc * inv + r_ref[...] + b_ref[...], 0.0)
        lg = jnp.dot(hh, fcw_ref[...], preferred_element_type=f32) + fcb_ref[...]
        mx = jnp.max(lg, axis=1, keepdims=True)
        lse = mx + jnp.log(jnp.sum(jnp.exp(lg - mx), axis=1, keepdims=True))
        o_ref[...] = lg - lse

    return pl.pallas_call(
        body,
        grid=(Np // blk,),
        in_specs=[pl.BlockSpec((blk, hp), lambda i: (i, 0)),
                  pl.BlockSpec((blk, hp), lambda i: (i, 0)),
                  pl.BlockSpec((blk, 1), lambda i: (i, 0)),
                  pl.BlockSpec((blk, cp), lambda i: (i, 0)),
                  pl.BlockSpec((1, cp), lambda i: (0, 0)),
                  pl.BlockSpec((cp, 2), lambda i: (0, 0)),
                  pl.BlockSpec((1, 2), lambda i: (0, 0))],
        out_specs=pl.BlockSpec((blk, 2), lambda i: (i, 0)),
        out_shape=jax.ShapeDtypeStruct((Np, 2), f32),
    )


def _gpar(ps):
    rows = []
    for p in ps:
        a = -0.5 / (p["sigma"] ** 2 + 1e-8)
        rows.append(jnp.concatenate([a, p["mu"]], axis=1))
    return jnp.concatenate(rows, axis=0)


def _wperm(W, root, cout):
    h = cout // 2
    order = np.array([k * cout + c * h + j
                      for c in range(2) for k in range(3) for j in range(h)])
    return jnp.concatenate([W[:, order], root], axis=1)


def kernel(x, edges_l6, ev6, edges_l5, ev5, pool_idx, unpool_idx, params):
    P = params
    src6 = edges_l6[0].astype(i32)
    dst6 = edges_l6[1].astype(i32)
    src5 = edges_l5[0].astype(i32)
    dst5 = edges_l5[1].astype(i32)

    cnt6r = _make_cnt(N6P, E6)(dst6)
    cnt6 = cnt6r[0, :, :1] + cnt6r[1, :, :1]
    cnt5r = _make_cnt(N5P, E5)(dst5)
    cnt5 = cnt5r[0, :, :1] + cnt5r[1, :, :1]

    g6 = _make_gk(E6, 4)(_gpar([P["c1"], P["c2"], P["c5"], P["c6"]]), ev6.T)
    g5 = _make_gk(E5, 2)(_gpar([P["c3"], P["c4"]]), ev5.T)
    g1, g2, gc5, gc6 = [g.reshape(-1) for g in g6]
    g3, g4 = [g.reshape(-1) for g in g5]

    agg6 = _make_agg(N6P, 64, E6, 128)
    agg5 = _make_agg(N5P, 128, E5, BE)

    xp = jnp.zeros((N6P, x.shape[1]), f32).at[:N6].set(x)
    b = {k: P[k]["b"].reshape(1, -1) for k in ("c1", "c2", "c3", "c4", "c5", "c6")}

    xw1, r1 = _make_mm(N6P, 32, 64)(xp, _wperm(P["c1"]["W"], P["c1"]["root"], 64))
    a1 = agg6(xw1.reshape(2 * N6P, -1), g1, src6, dst6)
    xw2, r2 = _make_mmfin(N6P, 64, 64)(a1[0], a1[1], cnt6, r1, b["c1"],
                                       _wperm(P["c2"]["W"], P["c2"]["root"], 64))
    a2 = agg6(xw2.reshape(2 * N6P, -1), g2, src6, dst6)
    skip = _make_fin(N6P, 64)(a2[0], a2[1], cnt6, r2, b["c2"])

    pidx = jnp.zeros((N5P * 7,), i32).at[:N5 * 7].set(
        pool_idx.astype(i32).reshape(-1))
    hp = _make_pool()(skip, pidx)

    xw3, r3 = _make_mm(N5P, 64, 128)(hp, _wperm(P["c3"]["W"], P["c3"]["root"], 128))
    a3 = agg5(xw3.reshape(2 * N5P, -1), g3, src5, dst5)
    xw4, r4 = _make_mmfin(N5P, 128, 128)(a3[0], a3[1], cnt5, r3, b["c3"],
                                         _wperm(P["c4"]["W"], P["c4"]["root"], 128))
    a4 = agg5(xw4.reshape(2 * N5P, -1), g4, src5, dst5)
    h4 = _make_fin(N5P, 128)(a4[0], a4[1], cnt5, r4, b["c4"])

    ar5 = jnp.arange(N5, dtype=i32)
    u0 = jnp.concatenate([ar5, unpool_idx[:, 0].astype(i32),
                          jnp.zeros((N6P - N6,), i32)])
    u1 = jnp.concatenate([ar5, unpool_idx[:, 1].astype(i32),
                          jnp.zeros((N6P - N6,), i32)])
    uf = jnp.stack([u0, u1], axis=1).reshape(-1)
    up = _make_unpool()(h4, uf)

    p5 = P["c5"]
    wa = _wperm(p5["W"][:128], p5["root"][:128], 64)
    wb = _wperm(p5["W"][128:], p5["root"][128:], 64)
    xw5, r5 = _make_mm2(N6P, 128, 64, 64)(up, skip, wa, wb)
    a5 = agg6(xw5.reshape(2 * N6P, -1), gc5, src6, dst6)
    xw6, r6 = _make_mmfin(N6P, 64, 64)(a5[0], a5[1], cnt6, r5, b["c5"],
                                       _wperm(P["c6"]["W"], P["c6"]["root"], 64))
    a6 = agg6(xw6.reshape(2 * N6P, -1), gc6, src6, dst6)

    out = _make_final(N6P)(a6[0], a6[1], cnt6, r6, b["c6"], P["fc_w"],
                           P["fc_b"].reshape(1, 2))
    return out[:N6]


# speedup vs baseline: 3.1976x; 1.0384x over previous
import functools

import numpy as np

import jax
import jax.numpy as jnp
from jax import lax
from jax.experimental import pallas as pl
from jax.experimental.pallas import tpu as pltpu
from jax.experimental.pallas import tpu_sc as plsc

N6, N5 = 40962, 10242
E6, E5 = 245760, 61440
N6P, N5P = 41472, 10752
BE = 64

f32 = jnp.float32
i32 = jnp.int32


def _sc_mesh():
    return plsc.VectorSubcoreMesh(core_axis_name="c", subcore_axis_name="s",
                                  num_cores=2, num_subcores=16)


@functools.cache
def _make_agg(Np, cout, E, be):
    h = cout // 2
    kch = 3 * h
    ept = E // 16
    nb = ept // be
    npt_sc = Np // 16
    zr = 48

    @functools.partial(
        pl.kernel, mesh=_sc_mesh(),
        out_type=jax.ShapeDtypeStruct((2, Np, h), f32),
        compiler_params=pltpu.CompilerParams(use_tc_tiling_on_sc=False),
        scratch_types=[
            pltpu.VMEM((be,), i32), pltpu.VMEM((be,), i32),
            pltpu.VMEM((be,), i32), pltpu.VMEM((be,), i32),
            pltpu.VMEM((be,), i32), pltpu.VMEM((be,), i32),
            pltpu.VMEM((3 * be,), f32), pltpu.VMEM((3 * be,), f32),
            pltpu.VMEM((3 * be,), f32), pltpu.VMEM((3 * be,), f32),
            pltpu.VMEM((be, kch), f32), pltpu.VMEM((be, kch), f32),
            pltpu.VMEM((be, h), f32), pltpu.VMEM((be, h), f32),
            pltpu.VMEM((zr, h), f32),
            pltpu.VMEM_SHARED((Np, h), f32),
            pltpu.SemaphoreType.DMA, pltpu.SemaphoreType.DMA,
            pltpu.SemaphoreType.DMA, pltpu.SemaphoreType.DMA,
            pltpu.SemaphoreType.DMA, pltpu.SemaphoreType.DMA,
        ],
    )
    def agg(xws, g3, srcs, dsts, out,
            ix0, ix1, dr0, dr1, dr2, dr3, gv0, gv1, gv2, gv3,
            rw0, rw1, mg0, mg1,
            zbuf, acc_sh, sm0, sm1, sg0, sg1, ss0, ss1):
        idxb, rwb, mgb = [ix0, ix1], [rw0, rw1], [mg0, mg1]
        gring = [gv0, gv1, gv2, gv3]
        dring = [dr0, dr1, dr2, dr3]
        smb, sgb, ssb = [sm0, sm1], [sg0, sg1], [ss0, ss1]
        c = lax.axis_index("c")
        s = lax.axis_index("s")
        ebase = s * ept
        n0sc = s * npt_sc
        cNp = c * Np

        def zstore(n, _):
            for cc in range(h // 16):
                zbuf[n, pl.ds(cc * 16, 16)] = jnp.zeros((16,), f32)
            return 0
        lax.fori_loop(0, zr, zstore, 0)

        def zcopy(zi, _):
            pltpu.sync_copy(zbuf, acc_sh.at[pl.ds(n0sc + zi * zr, zr)])
            return 0
        lax.fori_loop(0, npt_sc // zr, zcopy, 0)
        plsc.subcore_barrier()

        def issue_meta(bi, par, ring):
            eb = ebase + bi * be
            pltpu.async_copy(srcs.at[pl.ds(eb, be)], idxb[par], smb[par])
            pltpu.async_copy(dsts.at[pl.ds(eb, be)], dring[ring], smb[par])
            for kk in range(3):
                pltpu.async_copy(g3.at[pl.ds(kk * E + eb, be)],
                                 gring[ring].at[pl.ds(kk * be, be)], smb[par])

        def wait_meta(par, ring):
            pltpu.make_async_copy(srcs.at[pl.ds(0, be)], idxb[par],
                                  smb[par]).wait()
            pltpu.make_async_copy(dsts.at[pl.ds(0, be)], dring[ring],
                                  smb[par]).wait()
            for kk in range(3):
                pltpu.make_async_copy(g3.at[pl.ds(0, be)],
                                      gring[ring].at[pl.ds(kk * be, be)],
                                      smb[par]).wait()

        def compute(par, ring):
            def sub(sb, _):
                base = sb * 16
                g0 = gring[ring][pl.ds(base, 16)]
                g1 = gring[ring][pl.ds(be + base, 16)]
                g2 = gring[ring][pl.ds(2 * be + base, 16)]
                rows = rwb[par]
                msg = mgb[par]
                for j2 in range(16):
                    j = base + j2
                    a = g0[j2]
                    b2 = g1[j2]
                    cg = g2[j2]
                    for cc in range(h // 16):
                        o = cc * 16
                        msg[j, pl.ds(o, 16)] = (
                            rows[j, pl.ds(o, 16)] * a
                            + rows[j, pl.ds(h + o, 16)] * b2
                            + rows[j, pl.ds(2 * h + o, 16)] * cg)
                return 0
            lax.fori_loop(0, be // 16, sub, 0)

        issue_meta(0, 0, 0)

        def quad(qi, _):
            for q in range(4):
                b = qi * 4 + q
                par = q % 2
                opar = 1 - par

                @pl.when(b <= nb + 2)
                def _():
                    @pl.when(b >= 3)
                    def _():
                        pltpu.make_async_copy(
                            mgb[opar], acc_sh.at[dring[(q + 1) % 4]],
                            ssb[opar]).wait()

                    @pl.when(b < nb)
                    def _():
                        wait_meta(par, q)
                        for kk in range(be // 16):
                            idxb[par][pl.ds(kk * 16, 16)] = (
                                idxb[par][pl.ds(kk * 16, 16)] + cNp)
                        pltpu.async_copy(xws.at[idxb[par]], rwb[par], sgb[par])

                    @pl.when(jnp.logical_and(b >= 1, b <= nb))
                    def _():
                        pltpu.make_async_copy(xws.at[pl.ds(0, be)],
                                              rwb[opar], sgb[opar]).wait()

                    @pl.when(b + 1 < nb)
                    def _():
                        issue_meta(b + 1, opar, (q + 1) % 4)

                    @pl.when(jnp.logical_and(b >= 1, b <= nb))
                    def _():
                        compute(opar, (q + 3) % 4)
                        pltpu.async_copy(mgb[opar],
                                         acc_sh.at[dring[(q + 3) % 4]],
                                         ssb[opar], add=True)
            return 0
        lax.fori_loop(0, (nb + 3 + 3) // 4, quad, 0)

        plsc.subcore_barrier()
        pltpu.sync_copy(acc_sh.at[pl.ds(n0sc, npt_sc)],
                        out.at[c, pl.ds(n0sc, npt_sc)])

    return agg


@functools.cache
def _make_cnt(Np, E, be=128):
    ept = E // 32
    nb = ept // be
    npt_sc = Np // 16
    zr = 48

    @functools.partial(
        pl.kernel, mesh=_sc_mesh(),
        out_type=jax.ShapeDtypeStruct((2, Np, 16), f32),
        compiler_params=pltpu.CompilerParams(use_tc_tiling_on_sc=False),
        scratch_types=[
            pltpu.VMEM((be,), i32), pltpu.VMEM((be,), i32),
            pltpu.VMEM((be,), i32), pltpu.VMEM((be,), i32),
            pltpu.VMEM((be, 16), f32),
            pltpu.VMEM((zr, 16), f32),
            pltpu.VMEM_SHARED((Np, 16), f32),
            pltpu.SemaphoreType.DMA, pltpu.SemaphoreType.DMA,
            pltpu.SemaphoreType.DMA, pltpu.SemaphoreType.DMA,
        ],
    )
    def cntk(dsts, out, dr0, dr1, dr2, dr3, ones_v, zbuf, acc_sh,
             sm0, sm1, ss0, ss1):
        dring = [dr0, dr1, dr2, dr3]
        smb, ssb = [sm0, sm1], [ss0, ss1]
        c = lax.axis_index("c")
        s = lax.axis_index("s")
        wid = s * 2 + c
        ebase = wid * ept
        n0sc = s * npt_sc

        def fill(n, _):
            ones_v[n, pl.ds(0, 16)] = jnp.full((16,), 1.0, f32)
            for cc in range(1):
                pass
            zbuf[jnp.minimum(n, zr - 1), pl.ds(0, 16)] = jnp.zeros((16,), f32)
            return 0
        lax.fori_loop(0, be, fill, 0)

        def zcopy(zi, _):
            pltpu.sync_copy(zbuf, acc_sh.at[pl.ds(n0sc + zi * zr, zr)])
            return 0
        lax.fori_loop(0, npt_sc // zr, zcopy, 0)
        plsc.subcore_barrier()

        def issue_meta(bi, par, ring):
            pltpu.async_copy(dsts.at[pl.ds(ebase + bi * be, be)],
                             dring[ring], smb[par])

        issue_meta(0, 0, 0)

        def quad(qi, _):
            for q in range(4):
                b = qi * 4 + q
                par = q % 2

                @pl.when(b <= nb + 1)
                def _():
                    @pl.when(jnp.logical_and(b >= 2, b <= nb + 1))
                    def _():
                        pltpu.make_async_copy(ones_v,
                                              acc_sh.at[dring[(q + 2) % 4]],
                                              ssb[par]).wait()

                    @pl.when(b < nb)
                    def _():
                        pltpu.make_async_copy(dsts.at[pl.ds(0, be)],
                                              dring[q], smb[par]).wait()
                        pltpu.async_copy(ones_v, acc_sh.at[dring[q]],
                                         ssb[par], add=True)

                    @pl.when(b + 1 < nb)
                    def _():
                        issue_meta(b + 1, 1 - par, (q + 1) % 4)
            return 0
        lax.fori_loop(0, (nb + 2 + 3) // 4, quad, 0)

        plsc.subcore_barrier()
        pltpu.sync_copy(acc_sh.at[pl.ds(n0sc, npt_sc)],
                        out.at[c, pl.ds(n0sc, npt_sc)])

    return cntk


@functools.cache
def _make_pool():
    npt, C = N5P // 32, 64
    nbn = 16

    @functools.partial(
        pl.kernel, mesh=_sc_mesh(),
        out_type=jax.ShapeDtypeStruct((N5P, C), f32),
        compiler_params=pltpu.CompilerParams(use_tc_tiling_on_sc=False),
        scratch_types=[
            pltpu.VMEM((7 * nbn,), i32), pltpu.VMEM((7 * nbn,), i32),
            pltpu.VMEM((7 * nbn, C), f32), pltpu.VMEM((7 * nbn, C), f32),
            pltpu.VMEM((npt, C), f32),
            pltpu.SemaphoreType.DMA, pltpu.SemaphoreType.DMA,
            pltpu.SemaphoreType.DMA, pltpu.SemaphoreType.DMA,
        ],
    )
    def pool(skip, pidx, out, ix0, ix1, rw0, rw1, out_v, si0, si1, sg0, sg1):
        idxb, rwb = [ix0, ix1], [rw0, rw1]
        sib, sgb = [si0, si1], [sg0, sg1]
        wid = lax.axis_index("s") * 2 + lax.axis_index("c")
        n0 = wid * npt
        nb = npt // nbn

        def issue_idx(bi, par):
            pltpu.async_copy(pidx.at[pl.ds((n0 + bi * nbn) * 7, 7 * nbn)],
                             idxb[par], sib[par])

        issue_idx(0, 0)

        def pair(i, _):
            for par in range(2):
                b = i * 2 + par
                opar = 1 - par

                @pl.when(b <= nb)
                def _():
                    @pl.when(b < nb)
                    def _():
                        pltpu.make_async_copy(pidx.at[pl.ds(0, 7 * nbn)],
                                              idxb[par], sib[par]).wait()
                        pltpu.async_copy(skip.at[idxb[par]], rwb[par], sgb[par])

                    @pl.when(b >= 1)
                    def _():
                        pltpu.make_async_copy(skip.at[pl.ds(0, 7 * nbn)],
                                              rwb[opar], sgb[opar]).wait()

                    @pl.when(b + 1 < nb)
                    def _():
                        issue_idx(b + 1, opar)

                    @pl.when(b >= 1)
                    def _():
                        bm = b - 1
                        rows = rwb[opar]
                        for j in range(nbn):
                            for cc in range(C // 16):
                                o = cc * 16
                                v = rows[7 * j, pl.ds(o, 16)]
                                for t in range(1, 7):
                                    v = jnp.maximum(v, rows[7 * j + t,
                                                            pl.ds(o, 16)])
                                out_v[bm * nbn + j, pl.ds(o, 16)] = v
            return 0
        lax.fori_loop(0, (nb + 2) // 2, pair, 0)
        pltpu.sync_copy(out_v, out.at[pl.ds(n0, npt)])

    return pool


@functools.cache
def _make_unpool():
    npt, C = N6P // 32, 128
    nbr = 48

    @functools.partial(
        pl.kernel, mesh=_sc_mesh(),
        out_type=jax.ShapeDtypeStruct((N6P, C), f32),
        compiler_params=pltpu.CompilerParams(use_tc_tiling_on_sc=False),
        scratch_types=[
            pltpu.VMEM((2 * nbr,), i32), pltpu.VMEM((2 * nbr,), i32),
            pltpu.VMEM((2 * nbr, C), f32), pltpu.VMEM((2 * nbr, C), f32),
            pltpu.VMEM((nbr, C), f32), pltpu.VMEM((nbr, C), f32),
            pltpu.SemaphoreType.DMA, pltpu.SemaphoreType.DMA,
            pltpu.SemaphoreType.DMA, pltpu.SemaphoreType.DMA,
            pltpu.SemaphoreType.DMA, pltpu.SemaphoreType.DMA,
        ],
    )
    def unpool(h4, uf, out, ix0, ix1, rw0, rw1, ov0, ov1,
               si0, si1, sg0, sg1, so0, so1):
        idxb, rwb, ovb = [ix0, ix1], [rw0, rw1], [ov0, ov1]
        sib, sgb, sob = [si0, si1], [sg0, sg1], [so0, so1]
        wid = lax.axis_index("s") * 2 + lax.axis_index("c")
        n0 = wid * npt
        nb = npt // nbr

        def issue_idx(bi, par):
            pltpu.async_copy(uf.at[pl.ds((n0 + bi * nbr) * 2, 2 * nbr)],
                             idxb[par], sib[par])

        issue_idx(0, 0)

        def pair(i, _):
            for par in range(2):
                b = i * 2 + par
                opar = 1 - par

                @pl.when(b <= nb + 1)
                def _():
                    @pl.when(jnp.logical_and(b >= 2, b <= nb + 1))
                    def _():
                        pltpu.make_async_copy(ovb[par],
                                              out.at[pl.ds(0, nbr)],
                                              sob[par]).wait()

                    @pl.when(b < nb)
                    def _():
                        pltpu.make_async_copy(uf.at[pl.ds(0, 2 * nbr)],
                                              idxb[par], sib[par]).wait()
                        pltpu.async_copy(h4.at[idxb[par]], rwb[par], sgb[par])

                    @pl.when(jnp.logical_and(b >= 1, b <= nb))
                    def _():
                        pltpu.make_async_copy(h4.at[pl.ds(0, 2 * nbr)],
                                              rwb[opar], sgb[opar]).wait()

                    @pl.when(b + 1 < nb)
                    def _():
                        issue_idx(b + 1, opar)

                    @pl.when(jnp.logical_and(b >= 1, b <= nb))
                    def _():
                        bm = b - 1
                        rows = rwb[opar]

                        def row(j, _):
                            for cc in range(C // 16):
                                o = cc * 16
                                ovb[opar][j, pl.ds(o, 16)] = (
                                    rows[2 * j, pl.ds(o, 16)]
                                    + rows[2 * j + 1, pl.ds(o, 16)]) * 0.5
                            return 0
                        lax.fori_loop(0, nbr, row, 0)
                        pltpu.async_copy(ovb[opar],
                                         out.at[pl.ds(n0 + bm * nbr, nbr)],
                                         sob[opar])
            return 0
        lax.fori_loop(0, (nb + 3) // 2, pair, 0)

    return unpool


@functools.cache
def _make_gk(E, nconv, blk=2048):
    def body(par_ref, ev_ref, *out_refs):
        e0 = ev_ref[0:1, :]
        e1 = ev_ref[1:2, :]
        for ic in range(nconv):
            for k in range(3):
                a0 = par_ref[ic * 3 + k, 0]
                a1 = par_ref[ic * 3 + k, 1]
                d0 = e0 - par_ref[ic * 3 + k, 2]
                d1 = e1 - par_ref[ic * 3 + k, 3]
                out_refs[ic][k:k + 1, :] = jnp.exp(a0 * d0 * d0 + a1 * d1 * d1)

    return pl.pallas_call(
        body,
        grid=(pl.cdiv(E, blk),),
        in_specs=[pl.BlockSpec(memory_space=pltpu.SMEM),
                  pl.BlockSpec((2, blk), lambda i: (0, i))],
        out_specs=[pl.BlockSpec((3, blk), lambda i: (0, i))] * nconv,
        out_shape=[jax.ShapeDtypeStruct((3, E), f32)] * nconv,
    )


@functools.cache
def _make_mm(Np, cin, cout, blk=768):
    kch = 3 * (cout // 2)

    def body(x_ref, w_ref, xw_ref, r_ref):
        y = jnp.dot(x_ref[...], w_ref[...], preferred_element_type=f32)
        xw_ref[0] = y[:, :kch]
        xw_ref[1] = y[:, kch:2 * kch]
        r_ref[...] = y[:, 2 * kch:]

    return pl.pallas_call(
        body,
        grid=(Np // blk,),
        in_specs=[pl.BlockSpec((blk, cin), lambda i: (i, 0)),
                  pl.BlockSpec((cin, 4 * cout), lambda i: (0, 0))],
        out_specs=[pl.BlockSpec((2, blk, kch), lambda i: (0, i, 0)),
                   pl.BlockSpec((blk, cout), lambda i: (i, 0))],
        out_shape=[jax.ShapeDtypeStruct((2, Np, kch), f32),
                   jax.ShapeDtypeStruct((Np, cout), f32)],
    )


@functools.cache
def _make_mmfin(Np, cp, cout, blk=768):
    hp = cp // 2
    kch = 3 * (cout // 2)

    def body(a0_ref, a1_ref, cnt_ref, r_ref, b_ref, w_ref, xw_ref, r2_ref):
        acc = jnp.concatenate([a0_ref[...], a1_ref[...]], axis=1)
        inv = 1.0 / jnp.maximum(cnt_ref[...], 1.0)
        hh = jnp.maximum(acc * inv + r_ref[...] + b_ref[...], 0.0)
        y = jnp.dot(hh, w_ref[...], preferred_element_type=f32)
        xw_ref[0] = y[:, :kch]
        xw_ref[1] = y[:, kch:2 * kch]
        r2_ref[...] = y[:, 2 * kch:]

    return pl.pallas_call(
        body,
        grid=(Np // blk,),
        in_specs=[pl.BlockSpec((blk, hp), lambda i: (i, 0)),
                  pl.BlockSpec((blk, hp), lambda i: (i, 0)),
                  pl.BlockSpec((blk, 1), lambda i: (i, 0)),
                  pl.BlockSpec((blk, cp), lambda i: (i, 0)),
                  pl.BlockSpec((1, cp), lambda i: (0, 0)),
                  pl.BlockSpec((cp, 4 * cout), lambda i: (0, 0))],
        out_specs=[pl.BlockSpec((2, blk, kch), lambda i: (0, i, 0)),
                   pl.BlockSpec((blk, cout), lambda i: (i, 0))],
        out_shape=[jax.ShapeDtypeStruct((2, Np, kch), f32),
                   jax.ShapeDtypeStruct((Np, cout), f32)],
    )


@functools.cache
def _make_fin(Np, cp, blk=768):
    hp = cp // 2

    def body(a0_ref, a1_ref, cnt_ref, r_ref, b_ref, h_ref):
        acc = jnp.concatenate([a0_ref[...], a1_ref[...]], axis=1)
        inv = 1.0 / jnp.maximum(cnt_ref[...], 1.0)
        h_ref[...] = jnp.maximum(acc * inv + r_ref[...] + b_ref[...], 0.0)

    return pl.pallas_call(
        body,
        grid=(Np // blk,),
        in_specs=[pl.BlockSpec((blk, hp), lambda i: (i, 0)),
                  pl.BlockSpec((blk, hp), lambda i: (i, 0)),
                  pl.BlockSpec((blk, 1), lambda i: (i, 0)),
                  pl.BlockSpec((blk, cp), lambda i: (i, 0)),
                  pl.BlockSpec((1, cp), lambda i: (0, 0))],
        out_specs=pl.BlockSpec((blk, cp), lambda i: (i, 0)),
        out_shape=jax.ShapeDtypeStruct((Np, cp), f32),
    )


@functools.cache
def _make_mm2(Np, c1, c2, cout, blk=768):
    kch = 3 * (cout // 2)

    def body(a_ref, b_ref, wa_ref, wb_ref, xw_ref, r_ref):
        y = (jnp.dot(a_ref[...], wa_ref[...], preferred_element_type=f32)
             + jnp.dot(b_ref[...], wb_ref[...], preferred_element_type=f32))
        xw_ref[0] = y[:, :kch]
        xw_ref[1] = y[:, kch:2 * kch]
        r_ref[...] = y[:, 2 * kch:]

    return pl.pallas_call(
        body,
        grid=(Np // blk,),
        in_specs=[pl.BlockSpec((blk, c1), lambda i: (i, 0)),
                  pl.BlockSpec((blk, c2), lambda i: (i, 0)),
                  pl.BlockSpec((c1, 4 * cout), lambda i: (0, 0)),
                  pl.BlockSpec((c2, 4 * cout), lambda i: (0, 0))],
        out_specs=[pl.BlockSpec((2, blk, kch), lambda i: (0, i, 0)),
                   pl.BlockSpec((blk, cout), lambda i: (i, 0))],
        out_shape=[jax.ShapeDtypeStruct((2, Np, kch), f32),
                   jax.ShapeDtypeStruct((Np, cout), f32)],
    )


@functools.cache
def _make_final(Np, cp=64, blk=768):
    hp = cp // 2

    def body(a0_ref, a1_ref, cnt_ref, r_ref, b_ref, fcw_ref, fcb_ref, o_ref):
        acc = jnp.concatenate([a0_ref[...], a1_ref[...]], axis=1)
        inv = 1.0 / jnp.maximum(cnt_ref[...], 1.0)
        hh = jnp.maximum(acc * inv + r_ref[...] + b_ref[...], 0.0)
        lg = jnp.dot(hh, fcw_ref[...], preferred_element_type=f32) + fcb_ref[...]
        mx = jnp.max(lg, axis=1, keepdims=True)
        lse = mx + jnp.log(jnp.sum(jnp.exp(lg - mx), axis=1, keepdims=True))
        o_ref[...] = lg - lse

    return pl.pallas_call(
        body,
        grid=(Np // blk,),
        in_specs=[pl.BlockSpec((blk, hp), lambda i: (i, 0)),
                  pl.BlockSpec((blk, hp), lambda i: (i, 0)),
                  pl.BlockSpec((blk, 1), lambda i: (i, 0)),
                  pl.BlockSpec((blk, cp), lambda i: (i, 0)),
                  pl.BlockSpec((1, cp), lambda i: (0, 0)),
                  pl.BlockSpec((cp, 2), lambda i: (0, 0)),
                  pl.BlockSpec((1, 2), lambda i: (0, 0))],
        out_specs=pl.BlockSpec((blk, 2), lambda i: (i, 0)),
        out_shape=jax.ShapeDtypeStruct((Np, 2), f32),
    )


def _gpar(ps):
    rows = []
    for p in ps:
        a = -0.5 / (p["sigma"] ** 2 + 1e-8)
        rows.append(jnp.concatenate([a, p["mu"]], axis=1))
    return jnp.concatenate(rows, axis=0)


def _wperm(W, root, cout):
    h = cout // 2
    order = np.array([k * cout + c * h + j
                      for c in range(2) for k in range(3) for j in range(h)])
    return jnp.concatenate([W[:, order], root], axis=1)


def kernel(x, edges_l6, ev6, edges_l5, ev5, pool_idx, unpool_idx, params):
    P = params
    src6 = edges_l6[0].astype(i32)
    dst6 = edges_l6[1].astype(i32)
    src5 = edges_l5[0].astype(i32)
    dst5 = edges_l5[1].astype(i32)

    cnt6r = _make_cnt(N6P, E6)(dst6)
    cnt6 = cnt6r[0, :, :1] + cnt6r[1, :, :1]
    cnt5r = _make_cnt(N5P, E5)(dst5)
    cnt5 = cnt5r[0, :, :1] + cnt5r[1, :, :1]

    g6 = _make_gk(E6, 4)(_gpar([P["c1"], P["c2"], P["c5"], P["c6"]]), ev6.T)
    g5 = _make_gk(E5, 2)(_gpar([P["c3"], P["c4"]]), ev5.T)
    g1, g2, gc5, gc6 = [g.reshape(-1) for g in g6]
    g3, g4 = [g.reshape(-1) for g in g5]

    agg6 = _make_agg(N6P, 64, E6, 128)
    agg5 = _make_agg(N5P, 128, E5, BE)

    xp = jnp.zeros((N6P, x.shape[1]), f32).at[:N6].set(x)
    b = {k: P[k]["b"].reshape(1, -1) for k in ("c1", "c2", "c3", "c4", "c5", "c6")}

    xw1, r1 = _make_mm(N6P, 32, 64)(xp, _wperm(P["c1"]["W"], P["c1"]["root"], 64))
    a1 = agg6(xw1.reshape(2 * N6P, -1), g1, src6, dst6)
    xw2, r2 = _make_mmfin(N6P, 64, 64)(a1[0], a1[1], cnt6, r1, b["c1"],
                                       _wperm(P["c2"]["W"], P["c2"]["root"], 64))
    a2 = agg6(xw2.reshape(2 * N6P, -1), g2, src6, dst6)
    skip = _make_fin(N6P, 64)(a2[0], a2[1], cnt6, r2, b["c2"])

    pidx = jnp.zeros((N5P * 7,), i32).at[:N5 * 7].set(
        pool_idx.astype(i32).reshape(-1))
    hp = _make_pool()(skip, pidx)

    xw3, r3 = _make_mm(N5P, 64, 128)(hp, _wperm(P["c3"]["W"], P["c3"]["root"], 128))
    a3 = agg5(xw3.reshape(2 * N5P, -1), g3, src5, dst5)
    xw4, r4 = _make_mmfin(N5P, 128, 128)(a3[0], a3[1], cnt5, r3, b["c3"],
                                         _wperm(P["c4"]["W"], P["c4"]["root"], 128))
    a4 = agg5(xw4.reshape(2 * N5P, -1), g4, src5, dst5)
    h4 = _make_fin(N5P, 128)(a4[0], a4[1], cnt5, r4, b["c4"])

    ar5 = jnp.arange(N5, dtype=i32)
    u0 = jnp.concatenate([ar5, unpool_idx[:, 0].astype(i32),
                          jnp.zeros((N6P - N6,), i32)])
    u1 = jnp.concatenate([ar5, unpool_idx[:, 1].astype(i32),
                          jnp.zeros((N6P - N6,), i32)])
    uf = jnp.stack([u0, u1], axis=1).reshape(-1)
    up = _make_unpool()(h4, uf)

    p5 = P["c5"]
    wa = _wperm(p5["W"][:128], p5["root"][:128], 64)
    wb = _wperm(p5["W"][128:], p5["root"][128:], 64)
    xw5, r5 = _make_mm2(N6P, 128, 64, 64)(up, skip, wa, wb)
    a5 = agg6(xw5.reshape(2 * N6P, -1), gc5, src6, dst6)
    xw6, r6 = _make_mmfin(N6P, 64, 64)(a5[0], a5[1], cnt6, r5, b["c5"],
                                       _wperm(P["c6"]["W"], P["c6"]["root"], 64))
    a6 = agg6(xw6.reshape(2 * N6P, -1), gc6, src6, dst6)

    out = _make_final(N6P)(a6[0], a6[1], cnt6, r6, b["c6"], P["fc_w"],
                           P["fc_b"].reshape(1, 2))
    return out[:N6]
